# baseline (speedup 1.0000x reference)
import jax
import jax.numpy as jnp
import numpy as np
from jax import lax
from jax.experimental import pallas as pl
from jax.experimental.pallas import tpu as pltpu
from jax.experimental.pallas import tpu_sc as plsc

N = 10000
E = 320000
D = 128
MIN = 1e-15
MAXNORM = 1.0 - 4e-3

NC = 2
NS = 16
NW = NC * NS
EPW = E // NW
CH = 80
NCHUNK = EPW // CH

BN = 1000
BE = 4000
EL = E // D


def _artanh(x):
    x = jnp.clip(x, -1.0 + 1e-7, 1.0 - 1e-7)
    return 0.5 * jnp.log((1.0 + x) / (1.0 - x))


def _rnorm(x):
    return jnp.maximum(jnp.sqrt(jnp.sum(x * x, axis=-1, keepdims=True)), MIN)


def _proj(x):
    n = _rnorm(x)
    return jnp.where(n > MAXNORM, x / n * MAXNORM, x)


def _expmap0(u):
    un = _rnorm(u)
    return _proj(jnp.tanh(un) * u / un)


def _logmap0(p):
    pn = _rnorm(p)
    return p / pn * _artanh(pn)


def _mobius_add(x, y):
    x2 = jnp.sum(x * x, axis=-1, keepdims=True)
    y2 = jnp.sum(y * y, axis=-1, keepdims=True)
    xy = jnp.sum(x * y, axis=-1, keepdims=True)
    num = (1.0 + 2.0 * xy + y2) * x + (1.0 - x2) * y
    den = 1.0 + 2.0 * xy + x2 * y2
    return num / jnp.maximum(den, MIN)


def _dotT(a, b):
    return lax.dot_general(a, b, (((1,), (1,)), ((), ())),
                           preferred_element_type=jnp.float32)


def _tc1_body(h_ref, wlin_ref, bias_ref, w1a_ref, w1b_ref, b1_ref,
              x_ref, a_ref, b_ref):
    h = h_ref[...]
    x1 = _logmap0(h)
    xw = _dotT(x1, wlin_ref[...])
    xe = _expmap0(xw)
    hb = _expmap0(bias_ref[...])
    x = _mobius_add(xe, hb)
    x_ref[...] = x
    x_t = _logmap0(x)
    a_ref[...] = _dotT(x_t, w1a_ref[...]) + b1_ref[...]
    b_ref[...] = _dotT(x_t, w1b_ref[...])


def _tc1_call(h, W_lin, bias, W1a, W1b, b1):
    nspec = pl.BlockSpec((BN, D), lambda i: (i, 0))
    wspec = pl.BlockSpec((D, D), lambda i: (0, 0))
    vspec = pl.BlockSpec((1, D), lambda i: (0, 0))
    return pl.pallas_call(
        _tc1_body,
        grid=(N // BN,),
        in_specs=[nspec, wspec, vspec, wspec, wspec, vspec],
        out_specs=[nspec, nspec, nspec],
        out_shape=[jax.ShapeDtypeStruct((N, D), jnp.float32)] * 3,
    )(h, W_lin, bias, W1a, W1b, b1)


def _sc1_body(x_hbm, a_hbm, b_hbm, row_hbm, col_hbm,
              ab_hbm, scl_hbm,
              rowv, colv,
              b00, b01, b02, b03, b10, b11, b12, b13,
              scl0, scl1, sem0, sem1, wsem0, wsem1):
    bufs = ((b00, b01, b02, b03), (b10, b11, b12, b13))
    scls = (scl0, scl1)
    sems = (sem0, sem1)
    wsems = (wsem0, wsem1)
    wid = lax.axis_index("s") * NC + lax.axis_index("c")
    base = wid * EPW
    pltpu.sync_copy(row_hbm.at[pl.ds(base, EPW)], rowv)
    pltpu.sync_copy(col_hbm.at[pl.ds(base, EPW)], colv)

    def fire(k, s, first):
        if not first:
            pltpu.make_async_copy(bufs[s][2], ab_hbm.at[pl.ds(0, CH)],
                                  wsems[s]).wait()
            pltpu.make_async_copy(scls[s], scl_hbm.at[pl.ds(0, CH)],
                                  wsems[s]).wait()
        o = k * CH
        ri = rowv.at[pl.ds(o, CH)]
        ci = colv.at[pl.ds(o, CH)]
        pltpu.async_copy(x_hbm.at[ri], bufs[s][0], sems[s])
        pltpu.async_copy(x_hbm.at[ci], bufs[s][1], sems[s])
        pltpu.async_copy(a_hbm.at[ri], bufs[s][2], sems[s])
        pltpu.async_copy(b_hbm.at[ci], bufs[s][3], sems[s])

    def drain(k, s):
        for b in range(4):
            pltpu.make_async_copy(x_hbm.at[pl.ds(0, CH)], bufs[s][b],
                                  sems[s]).wait()

        def ebody(i, c, _s=s):
            b0, b1, b2, b3 = bufs[_s]
            z16 = jnp.zeros((16,), jnp.float32)
            dacc, xacc, yacc = z16, z16, z16
            for j in range(8):
                sl = pl.ds(j * 16, 16)
                v1 = b0[i, sl]
                v2 = b1[i, sl]
                dacc = dacc + v1 * v2
                xacc = xacc + v1 * v1
                yacc = yacc + v2 * v2
                b2[i, sl] = b2[i, sl] + b3[i, sl]
            scls[_s][i, pl.ds(0, 16)] = dacc
            scls[_s][i, pl.ds(16, 16)] = xacc
            scls[_s][i, pl.ds(32, 16)] = yacc
            return c

        lax.fori_loop(0, CH, ebody, 0)

        off = base + k * CH
        pltpu.async_copy(bufs[s][2], ab_hbm.at[pl.ds(off, CH)], wsems[s])
        pltpu.async_copy(scls[s], scl_hbm.at[pl.ds(off, CH)], wsems[s])

    fire(0, 0, True)
    fire(1, 1, True)

    def body(j, carry):
        k0 = 2 * j
        drain(k0, 0)
        fire(k0 + 2, 0, False)
        drain(k0 + 1, 1)
        fire(k0 + 3, 1, False)
        return carry

    lax.fori_loop(0, (NCHUNK - 3) // 2, body, 0)
    drain(NCHUNK - 3, 0)
    fire(NCHUNK - 1, 0, False)
    drain(NCHUNK - 2, 1)
    drain(NCHUNK - 1, 0)
    for s in range(2):
        pltpu.make_async_copy(bufs[s][2], ab_hbm.at[pl.ds(0, CH)],
                              wsems[s]).wait()
        pltpu.make_async_copy(scls[s], scl_hbm.at[pl.ds(0, CH)],
                              wsems[s]).wait()


def _sc1_call(x, A, B, row, col):
    mesh = plsc.VectorSubcoreMesh(core_axis_name="c", subcore_axis_name="s")
    f = pl.kernel(
        _sc1_body,
        mesh=mesh,
        out_type=[jax.ShapeDtypeStruct((E, D), jnp.float32),
                  jax.ShapeDtypeStruct((E, 64), jnp.float32)],
        scratch_types=[pltpu.VMEM((EPW,), jnp.int32),
                       pltpu.VMEM((EPW,), jnp.int32)]
                      + [pltpu.VMEM((CH, D), jnp.float32)] * 8
                      + [pltpu.VMEM((CH, 64), jnp.float32)] * 2
                      + [pltpu.SemaphoreType.DMA] * 4,
    )
    return f(x, A, B, row, col)


def _tc2p_body(p_ref, m_ref, o_ref):
    o_ref[...] = jnp.dot(p_ref[...], m_ref[...],
                         preferred_element_type=jnp.float32)


def _tc2p_call(prt2, M):
    BP = 8000
    return pl.pallas_call(
        _tc2p_body,
        grid=(E // 2 // BP,),
        in_specs=[pl.BlockSpec((BP, D), lambda i: (i, 0)),
                  pl.BlockSpec((D, 8), lambda i: (0, 0))],
        out_specs=pl.BlockSpec((BP, 8), lambda i: (i, 0)),
        out_shape=jax.ShapeDtypeStruct((E // 2, 8), jnp.float32),
    )(prt2, M)


def _tc2a_body(x2_ref, y2_ref, dt_ref, geo_ref, g0_ref, f0_ref):
    x2 = x2_ref[...]
    y2 = y2_ref[...]
    dot = dt_ref[...]
    alpha = 1.0 - 2.0 * dot + y2
    beta = 1.0 - x2
    num2 = alpha * alpha * x2 + beta * beta * y2 - 2.0 * alpha * beta * dot
    den = jnp.maximum(1.0 - 2.0 * dot + x2 * y2, MIN)
    subn = jnp.maximum(jnp.sqrt(jnp.maximum(num2, 0.0)) / den, MIN)
    w = _artanh(subn)
    geo_ref[...] = 4.0 * w * w
    scal0 = jnp.maximum(beta, MIN) * w / (den * subn) * 0.01
    g0_ref[...] = -scal0 * alpha
    f0_ref[...] = scal0 * beta


def _tc2a_call(x2, y2, dt):
    spec = pl.BlockSpec((EL, D), lambda: (0, 0))
    return pl.pallas_call(
        _tc2a_body,
        in_specs=[spec, spec, spec],
        out_specs=[spec, spec, spec],
        out_shape=[jax.ShapeDtypeStruct((EL, D), jnp.float32)] * 3,
    )(x2, y2, dt)


def _tc2b_body(ab_ref, geo_ref, d_ref, u_ref, v_ref, w2c_ref, s_ref):
    z = ab_ref[...] + d_ref[...] * u_ref[...] + geo_ref[...] * v_ref[...]
    mid = z * jax.nn.sigmoid(z)
    s_ref[...] = jnp.dot(mid, w2c_ref[...],
                         preferred_element_type=jnp.float32)


def _tc2b_call(ab, geo, distances, u, v, w2c):
    espec = pl.BlockSpec((BE, D), lambda i: (i, 0))
    sspec = pl.BlockSpec((BE, 1), lambda i: (i, 0))
    vspec = pl.BlockSpec((1, D), lambda i: (0, 0))
    cspec = pl.BlockSpec((D, 1), lambda i: (0, 0))
    return pl.pallas_call(
        _tc2b_body,
        grid=(E // BE,),
        in_specs=[espec, sspec, sspec, vspec, vspec, cspec],
        out_specs=sspec,
        out_shape=jax.ShapeDtypeStruct((E, 1), jnp.float32),
    )(ab, geo, distances, u, v, w2c)


def _tc2c_body(s_ref, em_ref, g0_ref, f0_ref, b2_ref, g_ref, f_ref):
    att = jax.nn.sigmoid(s_ref[...] + b2_ref[...]) * em_ref[...]
    g_ref[...] = att * g0_ref[...]
    f_ref[...] = att * f0_ref[...]


def _tc2c_call(s2, em2, g0, f0, b2):
    spec = pl.BlockSpec((EL, D), lambda: (0, 0))
    vspec = pl.BlockSpec((1, D), lambda: (0, 0))
    return pl.pallas_call(
        _tc2c_body,
        in_specs=[spec, spec, spec, spec, vspec],
        out_specs=[spec, spec],
        out_shape=[jax.ShapeDtypeStruct((EL, D), jnp.float32)] * 2,
    )(s2, em2, g0, f0, b2)


def _sc2_body(x_hbm, row_hbm, col_hbm, gf_hbm, zero_hbm, out_hbm,
              ri0, ri1, p10, p20, p11, p21, gf0, gf1, acc,
              sem0, sem1, isem0, isem1):
    cid = lax.axis_index("c")
    sid = lax.axis_index("s")
    wid = sid * NC + cid
    rib = (ri0, ri1)
    p1b = (p10, p11)
    p2b = (p20, p21)
    gfb = (gf0, gf1)
    sems = (sem0, sem1)
    isems = (isem0, isem1)
    rows = 624
    pltpu.sync_copy(zero_hbm.at[pl.ds(sid * rows, rows)],
                    acc.at[pl.ds(sid * rows, rows)])

    @pl.when(sid == 0)
    def _():
        pltpu.sync_copy(zero_hbm.at[pl.ds(NS * rows, N - NS * rows)],
                        acc.at[pl.ds(NS * rows, N - NS * rows)])

    plsc.subcore_barrier()

    base = wid * EPW

    def fire_idx(k, s):
        off = base + k * CH
        pltpu.async_copy(row_hbm.at[pl.ds(off, CH)], rib[s].at[0], isems[s])
        pltpu.async_copy(col_hbm.at[pl.ds(off, CH)], rib[s].at[1], isems[s])

    def wait_idx(s):
        pltpu.make_async_copy(row_hbm.at[pl.ds(0, CH)], rib[s].at[0],
                              isems[s]).wait()
        pltpu.make_async_copy(row_hbm.at[pl.ds(0, CH)], rib[s].at[1],
                              isems[s]).wait()

    def fire(k, s):
        wait_idx(s)
        pltpu.async_copy(x_hbm.at[rib[s].at[0]], p1b[s], sems[s])
        pltpu.async_copy(x_hbm.at[rib[s].at[1]], p2b[s], sems[s])
        pltpu.async_copy(gf_hbm.at[pl.ds(2 * (base + k * CH), 2 * CH)],
                         gfb[s].at[pl.ds(0, 2 * CH)], sems[s])

    def drain(k, s):
        pltpu.make_async_copy(x_hbm.at[pl.ds(0, CH)], p1b[s], sems[s]).wait()
        pltpu.make_async_copy(x_hbm.at[pl.ds(0, CH)], p2b[s], sems[s]).wait()
        pltpu.make_async_copy(gf_hbm.at[pl.ds(0, 2 * CH)],
                              gfb[s].at[pl.ds(0, 2 * CH)], sems[s]).wait()

        def mbody(i, c, _s=s):
            gv = gfb[_s][pl.ds(2 * i, 16)]
            gs = jnp.full((16,), gv[0], jnp.float32)
            fs = jnp.full((16,), gv[1], jnp.float32)
            for j in range(8):
                sl = pl.ds(j * 16, 16)
                p1b[_s][i, sl] = gs * p1b[_s][i, sl] + fs * p2b[_s][i, sl]
            return c

        lax.fori_loop(0, CH, mbody, 0)
        pltpu.sync_copy(p1b[s], acc.at[rib[s].at[0]], add=True)
        fire_idx(jnp.minimum(k + 2, NCHUNK - 1), s)

    fire_idx(0, 0)
    fire_idx(1, 1)
    fire(0, 0)

    def body(j, carry):
        k0 = 2 * j
        fire(k0 + 1, 1)
        drain(k0, 0)
        fire(k0 + 2, 0)
        drain(k0 + 1, 1)
        return carry

    lax.fori_loop(0, (NCHUNK - 1) // 2, body, 0)
    drain(NCHUNK - 1, 0)
    wait_idx(0)
    wait_idx(1)

    plsc.subcore_barrier()

    @pl.when(sid == 0)
    def _():
        pltpu.sync_copy(acc, out_hbm.at[cid])


def _sc2_call(x, row, col, gf, zero):
    mesh = plsc.VectorSubcoreMesh(core_axis_name="c", subcore_axis_name="s")
    f = pl.kernel(
        _sc2_body,
        mesh=mesh,
        out_type=jax.ShapeDtypeStruct((NC, N, D), jnp.float32),
        scratch_types=[pltpu.VMEM((2, CH), jnp.int32)] * 2
                      + [pltpu.VMEM((CH, D), jnp.float32)] * 4
                      + [pltpu.VMEM((2 * CH + 16,), jnp.float32)] * 2
                      + [pltpu.VMEM_SHARED((N, D), jnp.float32)]
                      + [pltpu.SemaphoreType.DMA] * 4,
    )
    return f(x, row, col, gf, zero)


def _tc4_body(p_ref, x_ref, g_ref, bta_ref, o_ref):
    out = p_ref[0] + p_ref[1]
    x = x_ref[...]
    un = _rnorm(out)
    x2 = jnp.sum(x * x, axis=-1, keepdims=True)
    lam = 2.0 / jnp.maximum(1.0 - x2, MIN)
    second = jnp.tanh(0.5 * lam * un) * out / un
    x1 = _proj(_mobius_add(x, second))
    t = _logmap0(x1)
    mu = jnp.mean(t, axis=-1, keepdims=True)
    var = jnp.mean((t - mu) ** 2, axis=-1, keepdims=True)
    t = (t - mu) / jnp.sqrt(var + 1e-5) * g_ref[...] + bta_ref[...]
    x2e = _expmap0(t)
    xt = _logmap0(x2e)
    xt = xt * jax.nn.sigmoid(xt)
    o_ref[...] = _expmap0(xt)


def _tc4_call(parts, x, ln_gamma, ln_beta):
    nspec = pl.BlockSpec((BN, D), lambda i: (i, 0))
    pspec = pl.BlockSpec((NC, BN, D), lambda i: (0, i, 0))
    vspec = pl.BlockSpec((1, D), lambda i: (0, 0))
    return pl.pallas_call(
        _tc4_body,
        grid=(N // BN,),
        in_specs=[pspec, nspec, vspec, vspec],
        out_specs=nspec,
        out_shape=jax.ShapeDtypeStruct((N, D), jnp.float32),
    )(parts, x, ln_gamma, ln_beta)


def kernel(h, distances, edges, node_mask, edge_mask, W_lin, bias,
           ln_gamma, ln_beta, att_W1, att_b1, att_W2, att_b2):
    row = edges[0]
    col = edges[1]
    W1a = att_W1[:, :D]
    W1b = att_W1[:, D:2 * D]
    u = att_W1[:, 2 * D].reshape(1, D)
    v = att_W1[:, 2 * D + 1].reshape(1, D)
    w2c = att_W2.reshape(D, 1)
    b1 = att_b1.reshape(1, D)
    b2 = jnp.broadcast_to(att_b2.reshape(1, 1), (1, D))

    m_np = np.zeros((D, 8), np.float32)
    for _h in (0, 1):
        m_np[64 * _h + 0:64 * _h + 16, 4 * _h + 0] = 1.0
        m_np[64 * _h + 16:64 * _h + 32, 4 * _h + 1] = 1.0
        m_np[64 * _h + 32:64 * _h + 48, 4 * _h + 2] = 1.0
    M = jnp.asarray(m_np)

    x, A, B = _tc1_call(h, W_lin, bias, W1a, W1b, b1)
    ab, scl = _sc1_call(x, A, B, row, col)
    s8 = _tc2p_call(scl.reshape(E // 2, D), M)
    dt2 = s8[:, 0::4].reshape(EL, D)
    x2d = s8[:, 1::4].reshape(EL, D)
    y2d = s8[:, 2::4].reshape(EL, D)
    geo2, g02, f02 = _tc2a_call(x2d, y2d, dt2)
    s = _tc2b_call(ab, geo2.reshape(E, 1), distances, u, v, w2c)
    g2, f2 = _tc2c_call(s.reshape(EL, D), edge_mask.reshape(EL, D), g02, f02, b2)
    gf = jnp.stack([g2.reshape(E), f2.reshape(E)], axis=-1).reshape(2 * E)
    zero = jnp.zeros((N, D), jnp.float32)
    parts = _sc2_call(x, row, col, gf, zero)
    xf = _tc4_call(parts, x, ln_gamma.reshape(1, D), ln_beta.reshape(1, D))
    return (xf, distances, edges, node_mask, edge_mask)


# speedup vs baseline: 1.5912x; 1.5912x over previous
import jax
import jax.numpy as jnp
import numpy as np
from jax import lax
from jax.experimental import pallas as pl
from jax.experimental.pallas import tpu as pltpu
from jax.experimental.pallas import tpu_sc as plsc

N = 10000
E = 320000
D = 128
MIN = 1e-15
MAXNORM = 1.0 - 4e-3

NC = 2
NS = 16
NW = NC * NS
EPW = E // NW
CH = 80
NCHUNK = EPW // CH

BN = 1000
BE = 4000
EL = E // D


def _artanh(x):
    x = jnp.clip(x, -1.0 + 1e-7, 1.0 - 1e-7)
    return 0.5 * jnp.log((1.0 + x) / (1.0 - x))


def _rnorm(x):
    return jnp.maximum(jnp.sqrt(jnp.sum(x * x, axis=-1, keepdims=True)), MIN)


def _proj(x):
    n = _rnorm(x)
    return jnp.where(n > MAXNORM, x / n * MAXNORM, x)


def _expmap0(u):
    un = _rnorm(u)
    return _proj(jnp.tanh(un) * u / un)


def _logmap0(p):
    pn = _rnorm(p)
    return p / pn * _artanh(pn)


def _mobius_add(x, y):
    x2 = jnp.sum(x * x, axis=-1, keepdims=True)
    y2 = jnp.sum(y * y, axis=-1, keepdims=True)
    xy = jnp.sum(x * y, axis=-1, keepdims=True)
    num = (1.0 + 2.0 * xy + y2) * x + (1.0 - x2) * y
    den = 1.0 + 2.0 * xy + x2 * y2
    return num / jnp.maximum(den, MIN)


def _dotT(a, b):
    return lax.dot_general(a, b, (((1,), (1,)), ((), ())),
                           preferred_element_type=jnp.float32)


def _tc1_body(h_ref, wlin_ref, bias_ref, w1a_ref, w1b_ref, b1_ref,
              x_ref, a_ref, b_ref):
    h = h_ref[...]
    x1 = _logmap0(h)
    xw = _dotT(x1, wlin_ref[...])
    xe = _expmap0(xw)
    hb = _expmap0(bias_ref[...])
    x = _mobius_add(xe, hb)
    x_ref[...] = x
    x_t = _logmap0(x)
    a_ref[...] = _dotT(x_t, w1a_ref[...]) + b1_ref[...]
    b_ref[...] = _dotT(x_t, w1b_ref[...])


def _tc1_call(h, W_lin, bias, W1a, W1b, b1):
    nspec = pl.BlockSpec((BN, D), lambda i: (i, 0))
    wspec = pl.BlockSpec((D, D), lambda i: (0, 0))
    vspec = pl.BlockSpec((1, D), lambda i: (0, 0))
    return pl.pallas_call(
        _tc1_body,
        grid=(N // BN,),
        in_specs=[nspec, wspec, vspec, wspec, wspec, vspec],
        out_specs=[nspec, nspec, nspec],
        out_shape=[jax.ShapeDtypeStruct((N, D), jnp.float32)] * 3,
    )(h, W_lin, bias, W1a, W1b, b1)


def _sc1_body(x_hbm, a_hbm, b_hbm, row_hbm, col_hbm,
              ab_hbm, scl_hbm,
              rowv, colv,
              b00, b01, b02, b03, b10, b11, b12, b13,
              scl0, scl1, sem0, sem1, wsem0, wsem1):
    bufs = ((b00, b01, b02, b03), (b10, b11, b12, b13))
    scls = (scl0, scl1)
    sems = (sem0, sem1)
    wsems = (wsem0, wsem1)
    wid = lax.axis_index("s") * NC + lax.axis_index("c")
    base = wid * EPW
    pltpu.sync_copy(row_hbm.at[pl.ds(base, EPW)], rowv)
    pltpu.sync_copy(col_hbm.at[pl.ds(base, EPW)], colv)

    def fire(k, s, first):
        if not first:
            pltpu.make_async_copy(bufs[s][2], ab_hbm.at[pl.ds(0, CH)],
                                  wsems[s]).wait()
            pltpu.make_async_copy(scls[s], scl_hbm.at[pl.ds(0, CH)],
                                  wsems[s]).wait()
        o = k * CH
        ri = rowv.at[pl.ds(o, CH)]
        ci = colv.at[pl.ds(o, CH)]
        pltpu.async_copy(x_hbm.at[ri], bufs[s][0], sems[s])
        pltpu.async_copy(x_hbm.at[ci], bufs[s][1], sems[s])
        pltpu.async_copy(a_hbm.at[ri], bufs[s][2], sems[s])
        pltpu.async_copy(b_hbm.at[ci], bufs[s][3], sems[s])

    def drain(k, s):
        for b in range(4):
            pltpu.make_async_copy(x_hbm.at[pl.ds(0, CH)], bufs[s][b],
                                  sems[s]).wait()

        def ebody(i, c, _s=s):
            b0, b1, b2, b3 = bufs[_s]
            z16 = jnp.zeros((16,), jnp.float32)
            dacc, xacc, yacc = z16, z16, z16
            for j in range(8):
                sl = pl.ds(j * 16, 16)
                v1 = b0[i, sl]
                v2 = b1[i, sl]
                dacc = dacc + v1 * v2
                xacc = xacc + v1 * v1
                yacc = yacc + v2 * v2
                b2[i, sl] = b2[i, sl] + b3[i, sl]
            scls[_s][i, pl.ds(0, 16)] = dacc
            scls[_s][i, pl.ds(16, 16)] = xacc
            scls[_s][i, pl.ds(32, 16)] = yacc
            return c

        lax.fori_loop(0, CH, ebody, 0)

        off = base + k * CH
        pltpu.async_copy(bufs[s][2], ab_hbm.at[pl.ds(off, CH)], wsems[s])
        pltpu.async_copy(scls[s], scl_hbm.at[pl.ds(off, CH)], wsems[s])

    fire(0, 0, True)
    fire(1, 1, True)

    def body(j, carry):
        k0 = 2 * j
        drain(k0, 0)
        fire(k0 + 2, 0, False)
        drain(k0 + 1, 1)
        fire(k0 + 3, 1, False)
        return carry

    lax.fori_loop(0, (NCHUNK - 3) // 2, body, 0)
    drain(NCHUNK - 3, 0)
    fire(NCHUNK - 1, 0, False)
    drain(NCHUNK - 2, 1)
    drain(NCHUNK - 1, 0)
    for s in range(2):
        pltpu.make_async_copy(bufs[s][2], ab_hbm.at[pl.ds(0, CH)],
                              wsems[s]).wait()
        pltpu.make_async_copy(scls[s], scl_hbm.at[pl.ds(0, CH)],
                              wsems[s]).wait()


def _sc1_call(x, A, B, row, col):
    mesh = plsc.VectorSubcoreMesh(core_axis_name="c", subcore_axis_name="s")
    f = pl.kernel(
        _sc1_body,
        mesh=mesh,
        out_type=[jax.ShapeDtypeStruct((E, D), jnp.float32),
                  jax.ShapeDtypeStruct((E, 64), jnp.float32)],
        scratch_types=[pltpu.VMEM((EPW,), jnp.int32),
                       pltpu.VMEM((EPW,), jnp.int32)]
                      + [pltpu.VMEM((CH, D), jnp.float32)] * 8
                      + [pltpu.VMEM((CH, 64), jnp.float32)] * 2
                      + [pltpu.SemaphoreType.DMA] * 4,
    )
    return f(x, A, B, row, col)


def _tc2_body(ab_ref, sp_ref, d_ref, em_ref, u_ref, v_ref, w2c_ref,
              b2_ref, m_ref, g_ref, f_ref):
    s8 = jnp.dot(sp_ref[...], m_ref[...], preferred_element_type=jnp.float32)
    dot = s8[:, 0:1]
    x2 = s8[:, 1:2]
    y2 = s8[:, 2:3]
    alpha = 1.0 - 2.0 * dot + y2
    beta = 1.0 - x2
    num2 = alpha * alpha * x2 + beta * beta * y2 - 2.0 * alpha * beta * dot
    den = jnp.maximum(1.0 - 2.0 * dot + x2 * y2, MIN)
    subn = jnp.maximum(jnp.sqrt(jnp.maximum(num2, 0.0)) / den, MIN)
    w = _artanh(subn)
    geo = 4.0 * w * w
    z = ab_ref[...] + d_ref[...] * u_ref[...] + geo * v_ref[...]
    mid = z * jax.nn.sigmoid(z)
    sv = jnp.dot(mid, w2c_ref[...],
                 preferred_element_type=jnp.float32) + b2_ref[0, 0]
    att = jax.nn.sigmoid(sv) * em_ref[...]
    scal = att * jnp.maximum(beta, MIN) * w / (den * subn) * 0.01
    g_ref[...] = -scal * alpha
    f_ref[...] = scal * beta


def _tc2_call(ab, scl, distances, edge_mask, u, v, w2c, b2, M64):
    espec = pl.BlockSpec((BE, D), lambda i: (i, 0))
    pspec = pl.BlockSpec((BE, 64), lambda i: (i, 0))
    sspec = pl.BlockSpec((BE, 1), lambda i: (i, 0))
    vspec = pl.BlockSpec((1, D), lambda i: (0, 0))
    cspec = pl.BlockSpec((D, 1), lambda i: (0, 0))
    mspec = pl.BlockSpec((64, 8), lambda i: (0, 0))
    return pl.pallas_call(
        _tc2_body,
        grid=(E // BE,),
        in_specs=[espec, pspec, sspec, sspec, vspec, vspec, cspec,
                  vspec, mspec],
        out_specs=[sspec, sspec],
        out_shape=[jax.ShapeDtypeStruct((E, 1), jnp.float32)] * 2,
    )(ab, scl, distances, edge_mask, u, v, w2c, b2, M64)


def _sc2_body(x_hbm, row_hbm, col_hbm, g_hbm, f_hbm, zero_hbm, out_hbm,
              ri0, ri1, p10, p20, p11, p21, gb0, gb1, fb0, fb1, acc,
              sem0, sem1, isem0, isem1):
    cid = lax.axis_index("c")
    sid = lax.axis_index("s")
    wid = sid * NC + cid
    rib = (ri0, ri1)
    p1b = (p10, p11)
    p2b = (p20, p21)
    gb = (gb0, gb1)
    fb = (fb0, fb1)
    sems = (sem0, sem1)
    isems = (isem0, isem1)
    rows = 624
    pltpu.sync_copy(zero_hbm.at[pl.ds(sid * rows, rows)],
                    acc.at[pl.ds(sid * rows, rows)])

    @pl.when(sid == 0)
    def _():
        pltpu.sync_copy(zero_hbm.at[pl.ds(NS * rows, N - NS * rows)],
                        acc.at[pl.ds(NS * rows, N - NS * rows)])

    plsc.subcore_barrier()

    base = wid * EPW

    def fire_idx(k, s):
        off = base + k * CH
        pltpu.async_copy(row_hbm.at[pl.ds(off, CH)], rib[s].at[0], isems[s])
        pltpu.async_copy(col_hbm.at[pl.ds(off, CH)], rib[s].at[1], isems[s])

    def wait_idx(s):
        pltpu.make_async_copy(row_hbm.at[pl.ds(0, CH)], rib[s].at[0],
                              isems[s]).wait()
        pltpu.make_async_copy(row_hbm.at[pl.ds(0, CH)], rib[s].at[1],
                              isems[s]).wait()

    def fire(k, s):
        wait_idx(s)
        pltpu.async_copy(x_hbm.at[rib[s].at[0]], p1b[s], sems[s])
        pltpu.async_copy(x_hbm.at[rib[s].at[1]], p2b[s], sems[s])
        o = base + k * CH
        pltpu.async_copy(g_hbm.at[pl.ds(o, CH)], gb[s].at[pl.ds(0, CH)],
                         sems[s])
        pltpu.async_copy(f_hbm.at[pl.ds(o, CH)], fb[s].at[pl.ds(0, CH)],
                         sems[s])

    def drain(k, s):
        pltpu.make_async_copy(x_hbm.at[pl.ds(0, CH)], p1b[s], sems[s]).wait()
        pltpu.make_async_copy(x_hbm.at[pl.ds(0, CH)], p2b[s], sems[s]).wait()
        pltpu.make_async_copy(g_hbm.at[pl.ds(0, CH)],
                              gb[s].at[pl.ds(0, CH)], sems[s]).wait()
        pltpu.make_async_copy(f_hbm.at[pl.ds(0, CH)],
                              fb[s].at[pl.ds(0, CH)], sems[s]).wait()

        def mbody(i, c, _s=s):
            gv = gb[_s][pl.ds(i, 16)]
            fv = fb[_s][pl.ds(i, 16)]
            gs = jnp.full((16,), gv[0], jnp.float32)
            fs = jnp.full((16,), fv[0], jnp.float32)
            for j in range(8):
                sl = pl.ds(j * 16, 16)
                p1b[_s][i, sl] = gs * p1b[_s][i, sl] + fs * p2b[_s][i, sl]
            return c

        lax.fori_loop(0, CH, mbody, 0)
        pltpu.sync_copy(p1b[s], acc.at[rib[s].at[0]], add=True)
        fire_idx(jnp.minimum(k + 2, NCHUNK - 1), s)

    fire_idx(0, 0)
    fire_idx(1, 1)
    fire(0, 0)

    def body(j, carry):
        k0 = 2 * j
        fire(k0 + 1, 1)
        drain(k0, 0)
        fire(k0 + 2, 0)
        drain(k0 + 1, 1)
        return carry

    lax.fori_loop(0, (NCHUNK - 1) // 2, body, 0)
    drain(NCHUNK - 1, 0)
    wait_idx(0)
    wait_idx(1)

    plsc.subcore_barrier()

    @pl.when(sid == 0)
    def _():
        pltpu.sync_copy(acc, out_hbm.at[cid])


def _sc2_call(x, row, col, g, f, zero):
    mesh = plsc.VectorSubcoreMesh(core_axis_name="c", subcore_axis_name="s")
    fn = pl.kernel(
        _sc2_body,
        mesh=mesh,
        out_type=jax.ShapeDtypeStruct((NC, N, D), jnp.float32),
        scratch_types=[pltpu.VMEM((2, CH), jnp.int32)] * 2
                      + [pltpu.VMEM((CH, D), jnp.float32)] * 4
                      + [pltpu.VMEM((CH + 16,), jnp.float32)] * 4
                      + [pltpu.VMEM_SHARED((N, D), jnp.float32)]
                      + [pltpu.SemaphoreType.DMA] * 4,
    )
    return fn(x, row, col, g, f, zero)


def _tc4_body(p_ref, x_ref, g_ref, bta_ref, o_ref):
    out = p_ref[0] + p_ref[1]
    x = x_ref[...]
    un = _rnorm(out)
    x2 = jnp.sum(x * x, axis=-1, keepdims=True)
    lam = 2.0 / jnp.maximum(1.0 - x2, MIN)
    second = jnp.tanh(0.5 * lam * un) * out / un
    x1 = _proj(_mobius_add(x, second))
    t = _logmap0(x1)
    mu = jnp.mean(t, axis=-1, keepdims=True)
    var = jnp.mean((t - mu) ** 2, axis=-1, keepdims=True)
    t = (t - mu) / jnp.sqrt(var + 1e-5) * g_ref[...] + bta_ref[...]
    x2e = _expmap0(t)
    xt = _logmap0(x2e)
    xt = xt * jax.nn.sigmoid(xt)
    o_ref[...] = _expmap0(xt)


def _tc4_call(parts, x, ln_gamma, ln_beta):
    nspec = pl.BlockSpec((BN, D), lambda i: (i, 0))
    pspec = pl.BlockSpec((NC, BN, D), lambda i: (0, i, 0))
    vspec = pl.BlockSpec((1, D), lambda i: (0, 0))
    return pl.pallas_call(
        _tc4_body,
        grid=(N // BN,),
        in_specs=[pspec, nspec, vspec, vspec],
        out_specs=nspec,
        out_shape=jax.ShapeDtypeStruct((N, D), jnp.float32),
    )(parts, x, ln_gamma, ln_beta)


def kernel(h, distances, edges, node_mask, edge_mask, W_lin, bias,
           ln_gamma, ln_beta, att_W1, att_b1, att_W2, att_b2):
    row = edges[0]
    col = edges[1]
    W1a = att_W1[:, :D]
    W1b = att_W1[:, D:2 * D]
    u = att_W1[:, 2 * D].reshape(1, D)
    v = att_W1[:, 2 * D + 1].reshape(1, D)
    w2c = att_W2.reshape(D, 1)
    b1 = att_b1.reshape(1, D)
    b2 = jnp.broadcast_to(att_b2.reshape(1, 1), (1, D))

    m_np = np.zeros((64, 8), np.float32)
    m_np[0:16, 0] = 1.0
    m_np[16:32, 1] = 1.0
    m_np[32:48, 2] = 1.0
    M64 = jnp.asarray(m_np)

    x, A, B = _tc1_call(h, W_lin, bias, W1a, W1b, b1)
    ab, scl = _sc1_call(x, A, B, row, col)
    g2, f2 = _tc2_call(ab, scl, distances, edge_mask, u, v, w2c, b2, M64)
    zero = jnp.zeros((N, D), jnp.float32)
    parts = _sc2_call(x, row, col, g2.reshape(E), f2.reshape(E), zero)
    xf = _tc4_call(parts, x, ln_gamma.reshape(1, D), ln_beta.reshape(1, D))
    return (xf, distances, edges, node_mask, edge_mask)


# speedup vs baseline: 1.6424x; 1.0322x over previous
import jax
import jax.numpy as jnp
import numpy as np
from jax import lax
from jax.experimental import pallas as pl
from jax.experimental.pallas import tpu as pltpu
from jax.experimental.pallas import tpu_sc as plsc

N = 10000
E = 320000
D = 128
MIN = 1e-15
MAXNORM = 1.0 - 4e-3

NC = 2
NS = 16
NW = NC * NS
EPW = E // NW
CH = 80
NCHUNK = EPW // CH

BN = 1000
BE = 4000
EL = E // D


def _artanh(x):
    x = jnp.clip(x, -1.0 + 1e-7, 1.0 - 1e-7)
    return 0.5 * jnp.log((1.0 + x) / (1.0 - x))


def _rnorm(x):
    return jnp.maximum(jnp.sqrt(jnp.sum(x * x, axis=-1, keepdims=True)), MIN)


def _proj(x):
    n = _rnorm(x)
    return jnp.where(n > MAXNORM, x / n * MAXNORM, x)


def _expmap0(u):
    un = _rnorm(u)
    return _proj(jnp.tanh(un) * u / un)


def _logmap0(p):
    pn = _rnorm(p)
    return p / pn * _artanh(pn)


def _mobius_add(x, y):
    x2 = jnp.sum(x * x, axis=-1, keepdims=True)
    y2 = jnp.sum(y * y, axis=-1, keepdims=True)
    xy = jnp.sum(x * y, axis=-1, keepdims=True)
    num = (1.0 + 2.0 * xy + y2) * x + (1.0 - x2) * y
    den = 1.0 + 2.0 * xy + x2 * y2
    return num / jnp.maximum(den, MIN)


def _dotT(a, b):
    return lax.dot_general(a, b, (((1,), (1,)), ((), ())),
                           preferred_element_type=jnp.float32)


def _tc1_body(h_ref, wlin_ref, bias_ref, w1a_ref, w1b_ref, b1_ref,
              x_ref, a_ref, b_ref):
    h = h_ref[...]
    x1 = _logmap0(h)
    xw = _dotT(x1, wlin_ref[...])
    xe = _expmap0(xw)
    hb = _expmap0(bias_ref[...])
    x = _mobius_add(xe, hb)
    x_ref[...] = x
    x_t = _logmap0(x)
    a_ref[...] = _dotT(x_t, w1a_ref[...]) + b1_ref[...]
    b_ref[...] = _dotT(x_t, w1b_ref[...])


def _tc1_call(h, W_lin, bias, W1a, W1b, b1):
    nspec = pl.BlockSpec((BN, D), lambda i: (i, 0))
    wspec = pl.BlockSpec((D, D), lambda i: (0, 0))
    vspec = pl.BlockSpec((1, D), lambda i: (0, 0))
    return pl.pallas_call(
        _tc1_body,
        grid=(N // BN,),
        in_specs=[nspec, wspec, vspec, wspec, wspec, vspec],
        out_specs=[nspec, nspec, nspec],
        out_shape=[jax.ShapeDtypeStruct((N, D), jnp.float32)] * 3,
    )(h, W_lin, bias, W1a, W1b, b1)


def _sc1_body(x_hbm, a_hbm, b_hbm, row_hbm, col_hbm,
              ab_hbm, scl_hbm,
              rowv, colv,
              b00, b01, b02, b03, b10, b11, b12, b13,
              scl0, scl1, sem0, sem1, wsem0, wsem1):
    bufs = ((b00, b01, b02, b03), (b10, b11, b12, b13))
    scls = (scl0, scl1)
    sems = (sem0, sem1)
    wsems = (wsem0, wsem1)
    wid = lax.axis_index("s") * NC + lax.axis_index("c")
    base = wid * EPW
    pltpu.sync_copy(row_hbm.at[pl.ds(base, EPW)], rowv)
    pltpu.sync_copy(col_hbm.at[pl.ds(base, EPW)], colv)

    def fire(k, s, first):
        if not first:
            pltpu.make_async_copy(bufs[s][2], ab_hbm.at[pl.ds(0, CH)],
                                  wsems[s]).wait()
            pltpu.make_async_copy(scls[s], scl_hbm.at[pl.ds(0, CH)],
                                  wsems[s]).wait()
        o = k * CH
        ri = rowv.at[pl.ds(o, CH)]
        ci = colv.at[pl.ds(o, CH)]
        pltpu.async_copy(x_hbm.at[ri], bufs[s][0], sems[s])
        pltpu.async_copy(x_hbm.at[ci], bufs[s][1], sems[s])
        pltpu.async_copy(a_hbm.at[ri], bufs[s][2], sems[s])
        pltpu.async_copy(b_hbm.at[ci], bufs[s][3], sems[s])

    def drain(k, s):
        for b in range(4):
            pltpu.make_async_copy(x_hbm.at[pl.ds(0, CH)], bufs[s][b],
                                  sems[s]).wait()

        def ebody(i, c, _s=s):
            b0, b1, b2, b3 = bufs[_s]
            z16 = jnp.zeros((16,), jnp.float32)
            dacc, xacc, yacc = z16, z16, z16
            for j in range(8):
                sl = pl.ds(j * 16, 16)
                v1 = b0[i, sl]
                v2 = b1[i, sl]
                dacc = dacc + v1 * v2
                xacc = xacc + v1 * v1
                yacc = yacc + v2 * v2
                b2[i, sl] = b2[i, sl] + b3[i, sl]
            scls[_s][i, pl.ds(0, 16)] = dacc
            scls[_s][i, pl.ds(16, 16)] = xacc
            scls[_s][i, pl.ds(32, 16)] = yacc
            return c

        lax.fori_loop(0, CH, ebody, 0)

        off = base + k * CH
        pltpu.async_copy(bufs[s][2], ab_hbm.at[pl.ds(off, CH)], wsems[s])
        pltpu.async_copy(scls[s], scl_hbm.at[pl.ds(off, CH)], wsems[s])

    fire(0, 0, True)
    fire(1, 1, True)

    def body(j, carry):
        k0 = 2 * j
        drain(k0, 0)
        fire(k0 + 2, 0, False)
        drain(k0 + 1, 1)
        fire(k0 + 3, 1, False)
        return carry

    lax.fori_loop(0, (NCHUNK - 3) // 2, body, 0)
    drain(NCHUNK - 3, 0)
    fire(NCHUNK - 1, 0, False)
    drain(NCHUNK - 2, 1)
    drain(NCHUNK - 1, 0)
    for s in range(2):
        pltpu.make_async_copy(bufs[s][2], ab_hbm.at[pl.ds(0, CH)],
                              wsems[s]).wait()
        pltpu.make_async_copy(scls[s], scl_hbm.at[pl.ds(0, CH)],
                              wsems[s]).wait()


def _sc1_call(x, A, B, row, col):
    mesh = plsc.VectorSubcoreMesh(core_axis_name="c", subcore_axis_name="s")
    f = pl.kernel(
        _sc1_body,
        mesh=mesh,
        out_type=[jax.ShapeDtypeStruct((E, D), jnp.float32),
                  jax.ShapeDtypeStruct((E, 64), jnp.float32)],
        scratch_types=[pltpu.VMEM((EPW,), jnp.int32),
                       pltpu.VMEM((EPW,), jnp.int32)]
                      + [pltpu.VMEM((CH, D), jnp.float32)] * 8
                      + [pltpu.VMEM((CH, 64), jnp.float32)] * 2
                      + [pltpu.SemaphoreType.DMA] * 4,
    )
    return f(x, A, B, row, col)


def _tc2_body(ab_ref, sp_ref, d_ref, em_ref, u_ref, v_ref, w2c_ref,
              b2_ref, m_ref, g_ref, f_ref):
    s8 = jnp.dot(sp_ref[...], m_ref[...], preferred_element_type=jnp.float32)
    dot = s8[:, 0:1]
    x2 = s8[:, 1:2]
    y2 = s8[:, 2:3]
    alpha = 1.0 - 2.0 * dot + y2
    beta = 1.0 - x2
    num2 = alpha * alpha * x2 + beta * beta * y2 - 2.0 * alpha * beta * dot
    den = jnp.maximum(1.0 - 2.0 * dot + x2 * y2, MIN)
    subn = jnp.maximum(jnp.sqrt(jnp.maximum(num2, 0.0)) / den, MIN)
    w = _artanh(subn)
    geo = 4.0 * w * w
    z = ab_ref[...] + d_ref[...] * u_ref[...] + geo * v_ref[...]
    mid = z * jax.nn.sigmoid(z)
    sv = jnp.dot(mid, w2c_ref[...],
                 preferred_element_type=jnp.float32) + b2_ref[0, 0]
    att = jax.nn.sigmoid(sv) * em_ref[...]
    scal = att * jnp.maximum(beta, MIN) * w / (den * subn) * 0.01
    g_ref[...] = -scal * alpha
    f_ref[...] = scal * beta


def _tc2_call(ab, scl, distances, edge_mask, u, v, w2c, b2, M64):
    espec = pl.BlockSpec((BE, D), lambda i: (i, 0))
    pspec = pl.BlockSpec((BE, 64), lambda i: (i, 0))
    sspec = pl.BlockSpec((BE, 1), lambda i: (i, 0))
    vspec = pl.BlockSpec((1, D), lambda i: (0, 0))
    cspec = pl.BlockSpec((D, 1), lambda i: (0, 0))
    mspec = pl.BlockSpec((64, 8), lambda i: (0, 0))
    return pl.pallas_call(
        _tc2_body,
        grid=(E // BE,),
        in_specs=[espec, pspec, sspec, sspec, vspec, vspec, cspec,
                  vspec, mspec],
        out_specs=[sspec, sspec],
        out_shape=[jax.ShapeDtypeStruct((E, 1), jnp.float32)] * 2,
    )(ab, scl, distances, edge_mask, u, v, w2c, b2, M64)


def _sc2_body(x_hbm, row_hbm, col_hbm, g_hbm, f_hbm, zero_hbm, out_hbm,
              ri0, ri1, sr0, sr1, p10, p20, p11, p21, gb0, gb1, fb0, fb1, acc,
              sem0, sem1, isem0, isem1):
    srow = (sr0, sr1)
    cid = lax.axis_index("c")
    sid = lax.axis_index("s")
    wid = sid * NC + cid
    rib = (ri0, ri1)
    p1b = (p10, p11)
    p2b = (p20, p21)
    gb = (gb0, gb1)
    fb = (fb0, fb1)
    sems = (sem0, sem1)
    isems = (isem0, isem1)
    rows = 624
    pltpu.sync_copy(zero_hbm.at[pl.ds(sid * rows, rows)],
                    acc.at[pl.ds(sid * rows, rows)])

    @pl.when(sid == 0)
    def _():
        pltpu.sync_copy(zero_hbm.at[pl.ds(NS * rows, N - NS * rows)],
                        acc.at[pl.ds(NS * rows, N - NS * rows)])

    plsc.subcore_barrier()

    base = wid * EPW

    def fire_idx(k, s):
        off = base + k * CH
        pltpu.async_copy(row_hbm.at[pl.ds(off, CH)], rib[s].at[0], isems[s])
        pltpu.async_copy(col_hbm.at[pl.ds(off, CH)], rib[s].at[1], isems[s])

    def wait_idx(s):
        pltpu.make_async_copy(row_hbm.at[pl.ds(0, CH)], rib[s].at[0],
                              isems[s]).wait()
        pltpu.make_async_copy(row_hbm.at[pl.ds(0, CH)], rib[s].at[1],
                              isems[s]).wait()

    def fire(k, s):
        wait_idx(s)
        pltpu.async_copy(x_hbm.at[rib[s].at[0]], p1b[s], sems[s])
        pltpu.async_copy(x_hbm.at[rib[s].at[1]], p2b[s], sems[s])
        o = base + k * CH
        pltpu.async_copy(g_hbm.at[pl.ds(o, CH)], gb[s].at[pl.ds(0, CH)],
                         sems[s])
        pltpu.async_copy(f_hbm.at[pl.ds(o, CH)], fb[s].at[pl.ds(0, CH)],
                         sems[s])

    def drain(k, s):
        pltpu.make_async_copy(x_hbm.at[pl.ds(0, CH)], p1b[s], sems[s]).wait()
        pltpu.make_async_copy(x_hbm.at[pl.ds(0, CH)], p2b[s], sems[s]).wait()
        pltpu.make_async_copy(g_hbm.at[pl.ds(0, CH)],
                              gb[s].at[pl.ds(0, CH)], sems[s]).wait()
        pltpu.make_async_copy(f_hbm.at[pl.ds(0, CH)],
                              fb[s].at[pl.ds(0, CH)], sems[s]).wait()
        for tt in range(CH // 16):
            srow[s][pl.ds(16 * tt, 16)] = rib[s][0, pl.ds(16 * tt, 16)]
        fire_idx(jnp.minimum(k + 2, NCHUNK - 1), s)

        def mbody(i, c, _s=s):
            gv = gb[_s][pl.ds(i, 16)]
            fv = fb[_s][pl.ds(i, 16)]
            gs = jnp.full((16,), gv[0], jnp.float32)
            fs = jnp.full((16,), fv[0], jnp.float32)
            for j in range(8):
                sl = pl.ds(j * 16, 16)
                p1b[_s][i, sl] = gs * p1b[_s][i, sl] + fs * p2b[_s][i, sl]
            return c

        lax.fori_loop(0, CH, mbody, 0)
        pltpu.sync_copy(p1b[s], acc.at[srow[s]], add=True)

    fire_idx(0, 0)
    fire_idx(1, 1)
    fire(0, 0)

    def body(j, carry):
        k0 = 2 * j
        fire(k0 + 1, 1)
        drain(k0, 0)
        fire(k0 + 2, 0)
        drain(k0 + 1, 1)
        return carry

    lax.fori_loop(0, (NCHUNK - 1) // 2, body, 0)
    drain(NCHUNK - 1, 0)
    wait_idx(0)
    wait_idx(1)

    plsc.subcore_barrier()

    @pl.when(sid == 0)
    def _():
        pltpu.sync_copy(acc, out_hbm.at[cid])


def _sc2_call(x, row, col, g, f, zero):
    mesh = plsc.VectorSubcoreMesh(core_axis_name="c", subcore_axis_name="s")
    fn = pl.kernel(
        _sc2_body,
        mesh=mesh,
        out_type=jax.ShapeDtypeStruct((NC, N, D), jnp.float32),
        scratch_types=[pltpu.VMEM((2, CH), jnp.int32)] * 2
                      + [pltpu.VMEM((CH,), jnp.int32)] * 2
                      + [pltpu.VMEM((CH, D), jnp.float32)] * 4
                      + [pltpu.VMEM((CH + 16,), jnp.float32)] * 4
                      + [pltpu.VMEM_SHARED((N, D), jnp.float32)]
                      + [pltpu.SemaphoreType.DMA] * 4,
    )
    return fn(x, row, col, g, f, zero)


def _tc4_body(p_ref, x_ref, g_ref, bta_ref, o_ref):
    out = p_ref[0] + p_ref[1]
    x = x_ref[...]
    un = _rnorm(out)
    x2 = jnp.sum(x * x, axis=-1, keepdims=True)
    lam = 2.0 / jnp.maximum(1.0 - x2, MIN)
    second = jnp.tanh(0.5 * lam * un) * out / un
    x1 = _proj(_mobius_add(x, second))
    t = _logmap0(x1)
    mu = jnp.mean(t, axis=-1, keepdims=True)
    var = jnp.mean((t - mu) ** 2, axis=-1, keepdims=True)
    t = (t - mu) / jnp.sqrt(var + 1e-5) * g_ref[...] + bta_ref[...]
    x2e = _expmap0(t)
    xt = _logmap0(x2e)
    xt = xt * jax.nn.sigmoid(xt)
    o_ref[...] = _expmap0(xt)


def _tc4_call(parts, x, ln_gamma, ln_beta):
    nspec = pl.BlockSpec((BN, D), lambda i: (i, 0))
    pspec = pl.BlockSpec((NC, BN, D), lambda i: (0, i, 0))
    vspec = pl.BlockSpec((1, D), lambda i: (0, 0))
    return pl.pallas_call(
        _tc4_body,
        grid=(N // BN,),
        in_specs=[pspec, nspec, vspec, vspec],
        out_specs=nspec,
        out_shape=jax.ShapeDtypeStruct((N, D), jnp.float32),
    )(parts, x, ln_gamma, ln_beta)


def kernel(h, distances, edges, node_mask, edge_mask, W_lin, bias,
           ln_gamma, ln_beta, att_W1, att_b1, att_W2, att_b2):
    row = edges[0]
    col = edges[1]
    W1a = att_W1[:, :D]
    W1b = att_W1[:, D:2 * D]
    u = att_W1[:, 2 * D].reshape(1, D)
    v = att_W1[:, 2 * D + 1].reshape(1, D)
    w2c = att_W2.reshape(D, 1)
    b1 = att_b1.reshape(1, D)
    b2 = jnp.broadcast_to(att_b2.reshape(1, 1), (1, D))

    m_np = np.zeros((64, 8), np.float32)
    m_np[0:16, 0] = 1.0
    m_np[16:32, 1] = 1.0
    m_np[32:48, 2] = 1.0
    M64 = jnp.asarray(m_np)

    x, A, B = _tc1_call(h, W_lin, bias, W1a, W1b, b1)
    ab, scl = _sc1_call(x, A, B, row, col)
    g2, f2 = _tc2_call(ab, scl, distances, edge_mask, u, v, w2c, b2, M64)
    zero = jnp.zeros((N, D), jnp.float32)
    parts = _sc2_call(x, row, col, g2.reshape(E), f2.reshape(E), zero)
    xf = _tc4_call(parts, x, ln_gamma.reshape(1, D), ln_beta.reshape(1, D))
    return (xf, distances, edges, node_mask, edge_mask)


# speedup vs baseline: 2.0393x; 1.2417x over previous
import jax
import jax.numpy as jnp
import numpy as np
from jax import lax
from jax.experimental import pallas as pl
from jax.experimental.pallas import tpu as pltpu
from jax.experimental.pallas import tpu_sc as plsc

N = 10000
E = 320000
D = 128
MIN = 1e-15
MAXNORM = 1.0 - 4e-3

NC = 2
NS = 16
NW = NC * NS
EPW = E // NW
CH = 80
NCHUNK = EPW // CH

BN = 1000
BE = 4000
EL = E // D


def _artanh(x):
    x = jnp.clip(x, -1.0 + 1e-7, 1.0 - 1e-7)
    return 0.5 * jnp.log((1.0 + x) / (1.0 - x))


def _rnorm(x):
    return jnp.maximum(jnp.sqrt(jnp.sum(x * x, axis=-1, keepdims=True)), MIN)


def _proj(x):
    n = _rnorm(x)
    return jnp.where(n > MAXNORM, x / n * MAXNORM, x)


def _expmap0(u):
    un = _rnorm(u)
    return _proj(jnp.tanh(un) * u / un)


def _logmap0(p):
    pn = _rnorm(p)
    return p / pn * _artanh(pn)


def _mobius_add(x, y):
    x2 = jnp.sum(x * x, axis=-1, keepdims=True)
    y2 = jnp.sum(y * y, axis=-1, keepdims=True)
    xy = jnp.sum(x * y, axis=-1, keepdims=True)
    num = (1.0 + 2.0 * xy + y2) * x + (1.0 - x2) * y
    den = 1.0 + 2.0 * xy + x2 * y2
    return num / jnp.maximum(den, MIN)


def _dotT(a, b):
    return lax.dot_general(a, b, (((1,), (1,)), ((), ())),
                           preferred_element_type=jnp.float32)


def _tc1_body(h_ref, wlin_ref, bias_ref, w1a_ref, w1b_ref, b1_ref,
              x_ref, a_ref, b_ref):
    h = h_ref[...]
    x1 = _logmap0(h)
    xw = _dotT(x1, wlin_ref[...])
    xe = _expmap0(xw)
    hb = _expmap0(bias_ref[...])
    x = _mobius_add(xe, hb)
    x_ref[...] = x
    x_t = _logmap0(x)
    a_ref[...] = _dotT(x_t, w1a_ref[...]) + b1_ref[...]
    b_ref[...] = _dotT(x_t, w1b_ref[...])


def _tc1_call(h, W_lin, bias, W1a, W1b, b1):
    nspec = pl.BlockSpec((BN, D), lambda i: (i, 0))
    wspec = pl.BlockSpec((D, D), lambda i: (0, 0))
    vspec = pl.BlockSpec((1, D), lambda i: (0, 0))
    return pl.pallas_call(
        _tc1_body,
        grid=(N // BN,),
        in_specs=[nspec, wspec, vspec, wspec, wspec, vspec],
        out_specs=[nspec, nspec, nspec],
        out_shape=[jax.ShapeDtypeStruct((N, D), jnp.float32)] * 3,
    )(h, W_lin, bias, W1a, W1b, b1)


def _sc1_body(x_hbm, a_hbm, b_hbm, row_hbm, col_hbm,
              ab_hbm, scl_hbm,
              rowv, colv,
              b00, b01, b02, b03, b10, b11, b12, b13,
              scl0, scl1, sem0, sem1, wsem0, wsem1):
    bufs = ((b00, b01, b02, b03), (b10, b11, b12, b13))
    scls = (scl0, scl1)
    sems = (sem0, sem1)
    wsems = (wsem0, wsem1)
    wid = lax.axis_index("s") * NC + lax.axis_index("c")
    base = wid * EPW
    pltpu.sync_copy(row_hbm.at[pl.ds(base, EPW)], rowv)
    pltpu.sync_copy(col_hbm.at[pl.ds(base, EPW)], colv)

    def fire(k, s, first):
        if not first:
            pltpu.make_async_copy(bufs[s][2], ab_hbm.at[pl.ds(0, CH)],
                                  wsems[s]).wait()
            pltpu.make_async_copy(scls[s], scl_hbm.at[pl.ds(0, CH)],
                                  wsems[s]).wait()
        o = k * CH
        ri = rowv.at[pl.ds(o, CH)]
        ci = colv.at[pl.ds(o, CH)]
        pltpu.async_copy(x_hbm.at[ri], bufs[s][0], sems[s])
        pltpu.async_copy(x_hbm.at[ci], bufs[s][1], sems[s])
        pltpu.async_copy(a_hbm.at[ri], bufs[s][2], sems[s])
        pltpu.async_copy(b_hbm.at[ci], bufs[s][3], sems[s])

    def drain(k, s):
        for b in range(4):
            pltpu.make_async_copy(x_hbm.at[pl.ds(0, CH)], bufs[s][b],
                                  sems[s]).wait()

        def ebody(i, _s=s):
            b0, b1, b2, b3 = bufs[_s]
            z16 = jnp.zeros((16,), jnp.float32)
            dacc, xacc, yacc = z16, z16, z16
            for j in range(8):
                sl = pl.ds(j * 16, 16)
                v1 = b0[i, sl]
                v2 = b1[i, sl]
                dacc = dacc + v1 * v2
                xacc = xacc + v1 * v1
                yacc = yacc + v2 * v2
                b2[i, sl] = b2[i, sl] + b3[i, sl]
            scls[_s][i, pl.ds(0, 16)] = dacc
            scls[_s][i, pl.ds(16, 16)] = xacc
            scls[_s][i, pl.ds(32, 16)] = yacc

        plsc.parallel_loop(0, CH, unroll=2)(ebody)

        off = base + k * CH
        pltpu.async_copy(bufs[s][2], ab_hbm.at[pl.ds(off, CH)], wsems[s])
        pltpu.async_copy(scls[s], scl_hbm.at[pl.ds(off, CH)], wsems[s])

    fire(0, 0, True)
    fire(1, 1, True)

    def body(j, carry):
        k0 = 2 * j
        drain(k0, 0)
        fire(k0 + 2, 0, False)
        drain(k0 + 1, 1)
        fire(k0 + 3, 1, False)
        return carry

    lax.fori_loop(0, (NCHUNK - 3) // 2, body, 0)
    drain(NCHUNK - 3, 0)
    fire(NCHUNK - 1, 0, False)
    drain(NCHUNK - 2, 1)
    drain(NCHUNK - 1, 0)
    for s in range(2):
        pltpu.make_async_copy(bufs[s][2], ab_hbm.at[pl.ds(0, CH)],
                              wsems[s]).wait()
        pltpu.make_async_copy(scls[s], scl_hbm.at[pl.ds(0, CH)],
                              wsems[s]).wait()


def _sc1_call(x, A, B, row, col):
    mesh = plsc.VectorSubcoreMesh(core_axis_name="c", subcore_axis_name="s")
    f = pl.kernel(
        _sc1_body,
        mesh=mesh,
        out_type=[jax.ShapeDtypeStruct((E, D), jnp.float32),
                  jax.ShapeDtypeStruct((E, 64), jnp.float32)],
        scratch_types=[pltpu.VMEM((EPW,), jnp.int32),
                       pltpu.VMEM((EPW,), jnp.int32)]
                      + [pltpu.VMEM((CH, D), jnp.float32)] * 8
                      + [pltpu.VMEM((CH, 64), jnp.float32)] * 2
                      + [pltpu.SemaphoreType.DMA] * 4,
    )
    return f(x, A, B, row, col)


def _tc2_body(ab_ref, sp_ref, d_ref, em_ref, u_ref, v_ref, w2c_ref,
              b2_ref, m_ref, g_ref, f_ref):
    s8 = jnp.dot(sp_ref[...], m_ref[...], preferred_element_type=jnp.float32)
    dot = s8[:, 0:1]
    x2 = s8[:, 1:2]
    y2 = s8[:, 2:3]
    alpha = 1.0 - 2.0 * dot + y2
    beta = 1.0 - x2
    num2 = alpha * alpha * x2 + beta * beta * y2 - 2.0 * alpha * beta * dot
    den = jnp.maximum(1.0 - 2.0 * dot + x2 * y2, MIN)
    subn = jnp.maximum(jnp.sqrt(jnp.maximum(num2, 0.0)) / den, MIN)
    w = _artanh(subn)
    geo = 4.0 * w * w
    z = ab_ref[...] + d_ref[...] * u_ref[...] + geo * v_ref[...]
    mid = z * jax.nn.sigmoid(z)
    sv = jnp.dot(mid, w2c_ref[...],
                 preferred_element_type=jnp.float32) + b2_ref[0, 0]
    att = jax.nn.sigmoid(sv) * em_ref[...]
    scal = att * jnp.maximum(beta, MIN) * w / (den * subn) * 0.01
    g_ref[...] = -scal * alpha
    f_ref[...] = scal * beta


def _tc2_call(ab, scl, distances, edge_mask, u, v, w2c, b2, M64):
    espec = pl.BlockSpec((BE, D), lambda i: (i, 0))
    pspec = pl.BlockSpec((BE, 64), lambda i: (i, 0))
    sspec = pl.BlockSpec((BE, 1), lambda i: (i, 0))
    vspec = pl.BlockSpec((1, D), lambda i: (0, 0))
    cspec = pl.BlockSpec((D, 1), lambda i: (0, 0))
    mspec = pl.BlockSpec((64, 8), lambda i: (0, 0))
    return pl.pallas_call(
        _tc2_body,
        grid=(E // BE,),
        in_specs=[espec, pspec, sspec, sspec, vspec, vspec, cspec,
                  vspec, mspec],
        out_specs=[sspec, sspec],
        out_shape=[jax.ShapeDtypeStruct((E, 1), jnp.float32)] * 2,
    )(ab, scl, distances, edge_mask, u, v, w2c, b2, M64)


def _sc2_body(x_hbm, row_hbm, col_hbm, g_hbm, f_hbm, zero_hbm, out_hbm,
              ri0, ri1, sr0, sr1, p10, p20, p11, p21, gb0, gb1, fb0, fb1, acc,
              sem0, sem1, isem0, isem1):
    srow = (sr0, sr1)
    cid = lax.axis_index("c")
    sid = lax.axis_index("s")
    wid = sid * NC + cid
    rib = (ri0, ri1)
    p1b = (p10, p11)
    p2b = (p20, p21)
    gb = (gb0, gb1)
    fb = (fb0, fb1)
    sems = (sem0, sem1)
    isems = (isem0, isem1)
    rows = 624
    pltpu.sync_copy(zero_hbm.at[pl.ds(sid * rows, rows)],
                    acc.at[pl.ds(sid * rows, rows)])

    @pl.when(sid == 0)
    def _():
        pltpu.sync_copy(zero_hbm.at[pl.ds(NS * rows, N - NS * rows)],
                        acc.at[pl.ds(NS * rows, N - NS * rows)])

    plsc.subcore_barrier()

    base = wid * EPW

    def fire_idx(k, s):
        off = base + k * CH
        pltpu.async_copy(row_hbm.at[pl.ds(off, CH)], rib[s].at[0], isems[s])
        pltpu.async_copy(col_hbm.at[pl.ds(off, CH)], rib[s].at[1], isems[s])

    def wait_idx(s):
        pltpu.make_async_copy(row_hbm.at[pl.ds(0, CH)], rib[s].at[0],
                              isems[s]).wait()
        pltpu.make_async_copy(row_hbm.at[pl.ds(0, CH)], rib[s].at[1],
                              isems[s]).wait()

    def fire(k, s):
        wait_idx(s)
        pltpu.async_copy(x_hbm.at[rib[s].at[0]], p1b[s], sems[s])
        pltpu.async_copy(x_hbm.at[rib[s].at[1]], p2b[s], sems[s])
        o = base + k * CH
        pltpu.async_copy(g_hbm.at[pl.ds(o, CH)], gb[s].at[pl.ds(0, CH)],
                         sems[s])
        pltpu.async_copy(f_hbm.at[pl.ds(o, CH)], fb[s].at[pl.ds(0, CH)],
                         sems[s])

    def drain(k, s):
        pltpu.make_async_copy(x_hbm.at[pl.ds(0, CH)], p1b[s], sems[s]).wait()
        pltpu.make_async_copy(x_hbm.at[pl.ds(0, CH)], p2b[s], sems[s]).wait()
        pltpu.make_async_copy(g_hbm.at[pl.ds(0, CH)],
                              gb[s].at[pl.ds(0, CH)], sems[s]).wait()
        pltpu.make_async_copy(f_hbm.at[pl.ds(0, CH)],
                              fb[s].at[pl.ds(0, CH)], sems[s]).wait()
        for tt in range(CH // 16):
            srow[s][pl.ds(16 * tt, 16)] = rib[s][0, pl.ds(16 * tt, 16)]
        fire_idx(jnp.minimum(k + 2, NCHUNK - 1), s)

        def mbody(i, _s=s):
            gv = gb[_s][pl.ds(i, 16)]
            fv = fb[_s][pl.ds(i, 16)]
            gs = jnp.full((16,), gv[0], jnp.float32)
            fs = jnp.full((16,), fv[0], jnp.float32)
            for j in range(8):
                sl = pl.ds(j * 16, 16)
                p1b[_s][i, sl] = gs * p1b[_s][i, sl] + fs * p2b[_s][i, sl]

        plsc.parallel_loop(0, CH, unroll=2)(mbody)
        pltpu.sync_copy(p1b[s], acc.at[srow[s]], add=True)

    fire_idx(0, 0)
    fire_idx(1, 1)
    fire(0, 0)

    def body(j, carry):
        k0 = 2 * j
        fire(k0 + 1, 1)
        drain(k0, 0)
        fire(k0 + 2, 0)
        drain(k0 + 1, 1)
        return carry

    lax.fori_loop(0, (NCHUNK - 1) // 2, body, 0)
    drain(NCHUNK - 1, 0)
    wait_idx(0)
    wait_idx(1)

    plsc.subcore_barrier()

    @pl.when(sid == 0)
    def _():
        pltpu.sync_copy(acc, out_hbm.at[cid])


def _sc2_call(x, row, col, g, f, zero):
    mesh = plsc.VectorSubcoreMesh(core_axis_name="c", subcore_axis_name="s")
    fn = pl.kernel(
        _sc2_body,
        mesh=mesh,
        out_type=jax.ShapeDtypeStruct((NC, N, D), jnp.float32),
        scratch_types=[pltpu.VMEM((2, CH), jnp.int32)] * 2
                      + [pltpu.VMEM((CH,), jnp.int32)] * 2
                      + [pltpu.VMEM((CH, D), jnp.float32)] * 4
                      + [pltpu.VMEM((CH + 16,), jnp.float32)] * 4
                      + [pltpu.VMEM_SHARED((N, D), jnp.float32)]
                      + [pltpu.SemaphoreType.DMA] * 4,
    )
    return fn(x, row, col, g, f, zero)


def _tc4_body(p_ref, x_ref, g_ref, bta_ref, o_ref):
    out = p_ref[0] + p_ref[1]
    x = x_ref[...]
    un = _rnorm(out)
    x2 = jnp.sum(x * x, axis=-1, keepdims=True)
    lam = 2.0 / jnp.maximum(1.0 - x2, MIN)
    second = jnp.tanh(0.5 * lam * un) * out / un
    x1 = _proj(_mobius_add(x, second))
    t = _logmap0(x1)
    mu = jnp.mean(t, axis=-1, keepdims=True)
    var = jnp.mean((t - mu) ** 2, axis=-1, keepdims=True)
    t = (t - mu) / jnp.sqrt(var + 1e-5) * g_ref[...] + bta_ref[...]
    x2e = _expmap0(t)
    xt = _logmap0(x2e)
    xt = xt * jax.nn.sigmoid(xt)
    o_ref[...] = _expmap0(xt)


def _tc4_call(parts, x, ln_gamma, ln_beta):
    nspec = pl.BlockSpec((BN, D), lambda i: (i, 0))
    pspec = pl.BlockSpec((NC, BN, D), lambda i: (0, i, 0))
    vspec = pl.BlockSpec((1, D), lambda i: (0, 0))
    return pl.pallas_call(
        _tc4_body,
        grid=(N // BN,),
        in_specs=[pspec, nspec, vspec, vspec],
        out_specs=nspec,
        out_shape=jax.ShapeDtypeStruct((N, D), jnp.float32),
    )(parts, x, ln_gamma, ln_beta)


def kernel(h, distances, edges, node_mask, edge_mask, W_lin, bias,
           ln_gamma, ln_beta, att_W1, att_b1, att_W2, att_b2):
    row = edges[0]
    col = edges[1]
    W1a = att_W1[:, :D]
    W1b = att_W1[:, D:2 * D]
    u = att_W1[:, 2 * D].reshape(1, D)
    v = att_W1[:, 2 * D + 1].reshape(1, D)
    w2c = att_W2.reshape(D, 1)
    b1 = att_b1.reshape(1, D)
    b2 = jnp.broadcast_to(att_b2.reshape(1, 1), (1, D))

    m_np = np.zeros((64, 8), np.float32)
    m_np[0:16, 0] = 1.0
    m_np[16:32, 1] = 1.0
    m_np[32:48, 2] = 1.0
    M64 = jnp.asarray(m_np)

    x, A, B = _tc1_call(h, W_lin, bias, W1a, W1b, b1)
    ab, scl = _sc1_call(x, A, B, row, col)
    g2, f2 = _tc2_call(ab, scl, distances, edge_mask, u, v, w2c, b2, M64)
    zero = jnp.zeros((N, D), jnp.float32)
    parts = _sc2_call(x, row, col, g2.reshape(E), f2.reshape(E), zero)
    xf = _tc4_call(parts, x, ln_gamma.reshape(1, D), ln_beta.reshape(1, D))
    return (xf, distances, edges, node_mask, edge_mask)


# speedup vs baseline: 2.0434x; 1.0020x over previous
import jax
import jax.numpy as jnp
import numpy as np
from jax import lax
from jax.experimental import pallas as pl
from jax.experimental.pallas import tpu as pltpu
from jax.experimental.pallas import tpu_sc as plsc

N = 10000
E = 320000
D = 128
MIN = 1e-15
MAXNORM = 1.0 - 4e-3

NC = 2
NS = 16
NW = NC * NS
EPW = E // NW
CH = 80
NCHUNK = EPW // CH

BN = 1000
BE = 4000
EL = E // D


def _artanh(x):
    x = jnp.clip(x, -1.0 + 1e-7, 1.0 - 1e-7)
    return 0.5 * jnp.log((1.0 + x) / (1.0 - x))


def _rnorm(x):
    return jnp.maximum(jnp.sqrt(jnp.sum(x * x, axis=-1, keepdims=True)), MIN)


def _proj(x):
    n = _rnorm(x)
    return jnp.where(n > MAXNORM, x / n * MAXNORM, x)


def _expmap0(u):
    un = _rnorm(u)
    return _proj(jnp.tanh(un) * u / un)


def _logmap0(p):
    pn = _rnorm(p)
    return p / pn * _artanh(pn)


def _mobius_add(x, y):
    x2 = jnp.sum(x * x, axis=-1, keepdims=True)
    y2 = jnp.sum(y * y, axis=-1, keepdims=True)
    xy = jnp.sum(x * y, axis=-1, keepdims=True)
    num = (1.0 + 2.0 * xy + y2) * x + (1.0 - x2) * y
    den = 1.0 + 2.0 * xy + x2 * y2
    return num / jnp.maximum(den, MIN)


def _dotT(a, b):
    return lax.dot_general(a, b, (((1,), (1,)), ((), ())),
                           preferred_element_type=jnp.float32)


def _tc1_body(h_ref, wlin_ref, bias_ref, w1a_ref, w1b_ref, b1_ref,
              x_ref, a_ref, b_ref):
    h = h_ref[...]
    x1 = _logmap0(h)
    xw = _dotT(x1, wlin_ref[...])
    xe = _expmap0(xw)
    hb = _expmap0(bias_ref[...])
    x = _mobius_add(xe, hb)
    x_ref[...] = x
    x_t = _logmap0(x)
    a_ref[...] = _dotT(x_t, w1a_ref[...]) + b1_ref[...]
    b_ref[...] = _dotT(x_t, w1b_ref[...])


def _tc1_call(h, W_lin, bias, W1a, W1b, b1):
    nspec = pl.BlockSpec((BN, D), lambda i: (i, 0))
    wspec = pl.BlockSpec((D, D), lambda i: (0, 0))
    vspec = pl.BlockSpec((1, D), lambda i: (0, 0))
    return pl.pallas_call(
        _tc1_body,
        grid=(N // BN,),
        in_specs=[nspec, wspec, vspec, wspec, wspec, vspec],
        out_specs=[nspec, nspec, nspec],
        out_shape=[jax.ShapeDtypeStruct((N, D), jnp.float32)] * 3,
    )(h, W_lin, bias, W1a, W1b, b1)


def _sc1_body(x_hbm, a_hbm, b_hbm, row_hbm, col_hbm,
              ab_hbm, scl_hbm,
              rowv, colv,
              b00, b01, b02, b03, b10, b11, b12, b13,
              scl0, scl1, sem0, sem1, wsem0, wsem1):
    bufs = ((b00, b01, b02, b03), (b10, b11, b12, b13))
    scls = (scl0, scl1)
    sems = (sem0, sem1)
    wsems = (wsem0, wsem1)
    wid = lax.axis_index("s") * NC + lax.axis_index("c")
    base = wid * EPW
    pltpu.sync_copy(row_hbm.at[pl.ds(base, EPW)], rowv)
    pltpu.sync_copy(col_hbm.at[pl.ds(base, EPW)], colv)

    def fire(k, s, first):
        if not first:
            pltpu.make_async_copy(bufs[s][2], ab_hbm.at[pl.ds(0, CH)],
                                  wsems[s]).wait()
            pltpu.make_async_copy(scls[s], scl_hbm.at[pl.ds(0, CH)],
                                  wsems[s]).wait()
        o = k * CH
        ri = rowv.at[pl.ds(o, CH)]
        ci = colv.at[pl.ds(o, CH)]
        pltpu.async_copy(x_hbm.at[ri], bufs[s][0], sems[s])
        pltpu.async_copy(x_hbm.at[ci], bufs[s][1], sems[s])
        pltpu.async_copy(a_hbm.at[ri], bufs[s][2], sems[s])
        pltpu.async_copy(b_hbm.at[ci], bufs[s][3], sems[s])

    def drain(k, s):
        for b in range(4):
            pltpu.make_async_copy(x_hbm.at[pl.ds(0, CH)], bufs[s][b],
                                  sems[s]).wait()

        def ebody(i, _s=s):
            b0, b1, b2, b3 = bufs[_s]
            z16 = jnp.zeros((16,), jnp.float32)
            dacc, xacc, yacc = z16, z16, z16
            for j in range(8):
                sl = pl.ds(j * 16, 16)
                v1 = b0[i, sl]
                v2 = b1[i, sl]
                dacc = dacc + v1 * v2
                xacc = xacc + v1 * v1
                yacc = yacc + v2 * v2
                b2[i, sl] = b2[i, sl] + b3[i, sl]
            scls[_s][i, pl.ds(0, 16)] = dacc
            scls[_s][i, pl.ds(16, 16)] = xacc
            scls[_s][i, pl.ds(32, 16)] = yacc

        plsc.parallel_loop(0, CH, unroll=4)(ebody)

        off = base + k * CH
        pltpu.async_copy(bufs[s][2], ab_hbm.at[pl.ds(off, CH)], wsems[s])
        pltpu.async_copy(scls[s], scl_hbm.at[pl.ds(off, CH)], wsems[s])

    fire(0, 0, True)
    fire(1, 1, True)

    def body(j, carry):
        k0 = 2 * j
        drain(k0, 0)
        fire(k0 + 2, 0, False)
        drain(k0 + 1, 1)
        fire(k0 + 3, 1, False)
        return carry

    lax.fori_loop(0, (NCHUNK - 3) // 2, body, 0)
    drain(NCHUNK - 3, 0)
    fire(NCHUNK - 1, 0, False)
    drain(NCHUNK - 2, 1)
    drain(NCHUNK - 1, 0)
    for s in range(2):
        pltpu.make_async_copy(bufs[s][2], ab_hbm.at[pl.ds(0, CH)],
                              wsems[s]).wait()
        pltpu.make_async_copy(scls[s], scl_hbm.at[pl.ds(0, CH)],
                              wsems[s]).wait()


def _sc1_call(x, A, B, row, col):
    mesh = plsc.VectorSubcoreMesh(core_axis_name="c", subcore_axis_name="s")
    f = pl.kernel(
        _sc1_body,
        mesh=mesh,
        out_type=[jax.ShapeDtypeStruct((E, D), jnp.float32),
                  jax.ShapeDtypeStruct((E, 64), jnp.float32)],
        scratch_types=[pltpu.VMEM((EPW,), jnp.int32),
                       pltpu.VMEM((EPW,), jnp.int32)]
                      + [pltpu.VMEM((CH, D), jnp.float32)] * 8
                      + [pltpu.VMEM((CH, 64), jnp.float32)] * 2
                      + [pltpu.SemaphoreType.DMA] * 4,
    )
    return f(x, A, B, row, col)


def _tc2_body(ab_ref, sp_ref, d_ref, em_ref, u_ref, v_ref, w2c_ref,
              b2_ref, m_ref, g_ref, f_ref):
    s8 = jnp.dot(sp_ref[...], m_ref[...], preferred_element_type=jnp.float32)
    dot = s8[:, 0:1]
    x2 = s8[:, 1:2]
    y2 = s8[:, 2:3]
    alpha = 1.0 - 2.0 * dot + y2
    beta = 1.0 - x2
    num2 = alpha * alpha * x2 + beta * beta * y2 - 2.0 * alpha * beta * dot
    den = jnp.maximum(1.0 - 2.0 * dot + x2 * y2, MIN)
    subn = jnp.maximum(jnp.sqrt(jnp.maximum(num2, 0.0)) / den, MIN)
    w = _artanh(subn)
    geo = 4.0 * w * w
    z = ab_ref[...] + d_ref[...] * u_ref[...] + geo * v_ref[...]
    mid = z * jax.nn.sigmoid(z)
    sv = jnp.dot(mid, w2c_ref[...],
                 preferred_element_type=jnp.float32) + b2_ref[0, 0]
    att = jax.nn.sigmoid(sv) * em_ref[...]
    scal = att * jnp.maximum(beta, MIN) * w / (den * subn) * 0.01
    g_ref[...] = -scal * alpha
    f_ref[...] = scal * beta


def _tc2_call(ab, scl, distances, edge_mask, u, v, w2c, b2, M64):
    espec = pl.BlockSpec((BE, D), lambda i: (i, 0))
    pspec = pl.BlockSpec((BE, 64), lambda i: (i, 0))
    sspec = pl.BlockSpec((BE, 1), lambda i: (i, 0))
    vspec = pl.BlockSpec((1, D), lambda i: (0, 0))
    cspec = pl.BlockSpec((D, 1), lambda i: (0, 0))
    mspec = pl.BlockSpec((64, 8), lambda i: (0, 0))
    return pl.pallas_call(
        _tc2_body,
        grid=(E // BE,),
        in_specs=[espec, pspec, sspec, sspec, vspec, vspec, cspec,
                  vspec, mspec],
        out_specs=[sspec, sspec],
        out_shape=[jax.ShapeDtypeStruct((E, 1), jnp.float32)] * 2,
    )(ab, scl, distances, edge_mask, u, v, w2c, b2, M64)


def _sc2_body(x_hbm, row_hbm, col_hbm, g_hbm, f_hbm, zero_hbm, out_hbm,
              ri0, ri1, sr0, sr1, p10, p20, p11, p21, gb0, gb1, fb0, fb1, acc,
              sem0, sem1, isem0, isem1):
    srow = (sr0, sr1)
    cid = lax.axis_index("c")
    sid = lax.axis_index("s")
    wid = sid * NC + cid
    rib = (ri0, ri1)
    p1b = (p10, p11)
    p2b = (p20, p21)
    gb = (gb0, gb1)
    fb = (fb0, fb1)
    sems = (sem0, sem1)
    isems = (isem0, isem1)
    rows = 624
    pltpu.sync_copy(zero_hbm.at[pl.ds(sid * rows, rows)],
                    acc.at[pl.ds(sid * rows, rows)])

    @pl.when(sid == 0)
    def _():
        pltpu.sync_copy(zero_hbm.at[pl.ds(NS * rows, N - NS * rows)],
                        acc.at[pl.ds(NS * rows, N - NS * rows)])

    plsc.subcore_barrier()

    base = wid * EPW

    def fire_idx(k, s):
        off = base + k * CH
        pltpu.async_copy(row_hbm.at[pl.ds(off, CH)], rib[s].at[0], isems[s])
        pltpu.async_copy(col_hbm.at[pl.ds(off, CH)], rib[s].at[1], isems[s])

    def wait_idx(s):
        pltpu.make_async_copy(row_hbm.at[pl.ds(0, CH)], rib[s].at[0],
                              isems[s]).wait()
        pltpu.make_async_copy(row_hbm.at[pl.ds(0, CH)], rib[s].at[1],
                              isems[s]).wait()

    def fire(k, s):
        wait_idx(s)
        pltpu.async_copy(x_hbm.at[rib[s].at[0]], p1b[s], sems[s])
        pltpu.async_copy(x_hbm.at[rib[s].at[1]], p2b[s], sems[s])
        o = base + k * CH
        pltpu.async_copy(g_hbm.at[pl.ds(o, CH)], gb[s].at[pl.ds(0, CH)],
                         sems[s])
        pltpu.async_copy(f_hbm.at[pl.ds(o, CH)], fb[s].at[pl.ds(0, CH)],
                         sems[s])

    def drain(k, s):
        pltpu.make_async_copy(x_hbm.at[pl.ds(0, CH)], p1b[s], sems[s]).wait()
        pltpu.make_async_copy(x_hbm.at[pl.ds(0, CH)], p2b[s], sems[s]).wait()
        pltpu.make_async_copy(g_hbm.at[pl.ds(0, CH)],
                              gb[s].at[pl.ds(0, CH)], sems[s]).wait()
        pltpu.make_async_copy(f_hbm.at[pl.ds(0, CH)],
                              fb[s].at[pl.ds(0, CH)], sems[s]).wait()
        for tt in range(CH // 16):
            srow[s][pl.ds(16 * tt, 16)] = rib[s][0, pl.ds(16 * tt, 16)]
        fire_idx(jnp.minimum(k + 2, NCHUNK - 1), s)

        def mbody(i, _s=s):
            gv = gb[_s][pl.ds(i, 16)]
            fv = fb[_s][pl.ds(i, 16)]
            gs = jnp.full((16,), gv[0], jnp.float32)
            fs = jnp.full((16,), fv[0], jnp.float32)
            for j in range(8):
                sl = pl.ds(j * 16, 16)
                p1b[_s][i, sl] = gs * p1b[_s][i, sl] + fs * p2b[_s][i, sl]

        plsc.parallel_loop(0, CH, unroll=4)(mbody)
        pltpu.sync_copy(p1b[s], acc.at[srow[s]], add=True)

    fire_idx(0, 0)
    fire_idx(1, 1)
    fire(0, 0)

    def body(j, carry):
        k0 = 2 * j
        fire(k0 + 1, 1)
        drain(k0, 0)
        fire(k0 + 2, 0)
        drain(k0 + 1, 1)
        return carry

    lax.fori_loop(0, (NCHUNK - 1) // 2, body, 0)
    drain(NCHUNK - 1, 0)
    wait_idx(0)
    wait_idx(1)

    plsc.subcore_barrier()

    @pl.when(sid == 0)
    def _():
        pltpu.sync_copy(acc, out_hbm.at[cid])


def _sc2_call(x, row, col, g, f, zero):
    mesh = plsc.VectorSubcoreMesh(core_axis_name="c", subcore_axis_name="s")
    fn = pl.kernel(
        _sc2_body,
        mesh=mesh,
        out_type=jax.ShapeDtypeStruct((NC, N, D), jnp.float32),
        scratch_types=[pltpu.VMEM((2, CH), jnp.int32)] * 2
                      + [pltpu.VMEM((CH,), jnp.int32)] * 2
                      + [pltpu.VMEM((CH, D), jnp.float32)] * 4
                      + [pltpu.VMEM((CH + 16,), jnp.float32)] * 4
                      + [pltpu.VMEM_SHARED((N, D), jnp.float32)]
                      + [pltpu.SemaphoreType.DMA] * 4,
    )
    return fn(x, row, col, g, f, zero)


def _tc4_body(p_ref, x_ref, g_ref, bta_ref, o_ref):
    out = p_ref[0] + p_ref[1]
    x = x_ref[...]
    un = _rnorm(out)
    x2 = jnp.sum(x * x, axis=-1, keepdims=True)
    lam = 2.0 / jnp.maximum(1.0 - x2, MIN)
    second = jnp.tanh(0.5 * lam * un) * out / un
    x1 = _proj(_mobius_add(x, second))
    t = _logmap0(x1)
    mu = jnp.mean(t, axis=-1, keepdims=True)
    var = jnp.mean((t - mu) ** 2, axis=-1, keepdims=True)
    t = (t - mu) / jnp.sqrt(var + 1e-5) * g_ref[...] + bta_ref[...]
    x2e = _expmap0(t)
    xt = _logmap0(x2e)
    xt = xt * jax.nn.sigmoid(xt)
    o_ref[...] = _expmap0(xt)


def _tc4_call(parts, x, ln_gamma, ln_beta):
    nspec = pl.BlockSpec((BN, D), lambda i: (i, 0))
    pspec = pl.BlockSpec((NC, BN, D), lambda i: (0, i, 0))
    vspec = pl.BlockSpec((1, D), lambda i: (0, 0))
    return pl.pallas_call(
        _tc4_body,
        grid=(N // BN,),
        in_specs=[pspec, nspec, vspec, vspec],
        out_specs=nspec,
        out_shape=jax.ShapeDtypeStruct((N, D), jnp.float32),
    )(parts, x, ln_gamma, ln_beta)


def kernel(h, distances, edges, node_mask, edge_mask, W_lin, bias,
           ln_gamma, ln_beta, att_W1, att_b1, att_W2, att_b2):
    row = edges[0]
    col = edges[1]
    W1a = att_W1[:, :D]
    W1b = att_W1[:, D:2 * D]
    u = att_W1[:, 2 * D].reshape(1, D)
    v = att_W1[:, 2 * D + 1].reshape(1, D)
    w2c = att_W2.reshape(D, 1)
    b1 = att_b1.reshape(1, D)
    b2 = jnp.broadcast_to(att_b2.reshape(1, 1), (1, D))

    m_np = np.zeros((64, 8), np.float32)
    m_np[0:16, 0] = 1.0
    m_np[16:32, 1] = 1.0
    m_np[32:48, 2] = 1.0
    M64 = jnp.asarray(m_np)

    x, A, B = _tc1_call(h, W_lin, bias, W1a, W1b, b1)
    ab, scl = _sc1_call(x, A, B, row, col)
    g2, f2 = _tc2_call(ab, scl, distances, edge_mask, u, v, w2c, b2, M64)
    zero = jnp.zeros((N, D), jnp.float32)
    parts = _sc2_call(x, row, col, g2.reshape(E), f2.reshape(E), zero)
    xf = _tc4_call(parts, x, ln_gamma.reshape(1, D), ln_beta.reshape(1, D))
    return (xf, distances, edges, node_mask, edge_mask)


# speedup vs baseline: 2.3999x; 1.1745x over previous
import jax
import jax.numpy as jnp
import numpy as np
from jax import lax
from jax.experimental import pallas as pl
from jax.experimental.pallas import tpu as pltpu
from jax.experimental.pallas import tpu_sc as plsc

N = 10000
E = 320000
D = 128
MIN = 1e-15
MAXNORM = 1.0 - 4e-3

NC = 2
NS = 16
NW = NC * NS
EPW = E // NW
CH = 80
NCHUNK = EPW // CH

BN = 1000
BE = 4000
EL = E // D


def _artanh(x):
    x = jnp.clip(x, -1.0 + 1e-7, 1.0 - 1e-7)
    return 0.5 * jnp.log((1.0 + x) / (1.0 - x))


def _rnorm(x):
    return jnp.maximum(jnp.sqrt(jnp.sum(x * x, axis=-1, keepdims=True)), MIN)


def _proj(x):
    n = _rnorm(x)
    return jnp.where(n > MAXNORM, x / n * MAXNORM, x)


def _expmap0(u):
    un = _rnorm(u)
    return _proj(jnp.tanh(un) * u / un)


def _logmap0(p):
    pn = _rnorm(p)
    return p / pn * _artanh(pn)


def _mobius_add(x, y):
    x2 = jnp.sum(x * x, axis=-1, keepdims=True)
    y2 = jnp.sum(y * y, axis=-1, keepdims=True)
    xy = jnp.sum(x * y, axis=-1, keepdims=True)
    num = (1.0 + 2.0 * xy + y2) * x + (1.0 - x2) * y
    den = 1.0 + 2.0 * xy + x2 * y2
    return num / jnp.maximum(den, MIN)


def _dotT(a, b):
    return lax.dot_general(a, b, (((1,), (1,)), ((), ())),
                           preferred_element_type=jnp.float32)


def _tc1_body(h_ref, wlin_ref, bias_ref, w1a_ref, w1b_ref, b1_ref,
              x_ref, a_ref, b_ref):
    h = h_ref[...]
    x1 = _logmap0(h)
    xw = _dotT(x1, wlin_ref[...])
    xe = _expmap0(xw)
    hb = _expmap0(bias_ref[...])
    x = _mobius_add(xe, hb)
    x_ref[...] = x
    x_t = _logmap0(x)
    a_ref[...] = _dotT(x_t, w1a_ref[...]) + b1_ref[...]
    b_ref[...] = _dotT(x_t, w1b_ref[...])


def _tc1_call(h, W_lin, bias, W1a, W1b, b1):
    nspec = pl.BlockSpec((BN, D), lambda i: (i, 0))
    wspec = pl.BlockSpec((D, D), lambda i: (0, 0))
    vspec = pl.BlockSpec((1, D), lambda i: (0, 0))
    return pl.pallas_call(
        _tc1_body,
        grid=(N // BN,),
        in_specs=[nspec, wspec, vspec, wspec, wspec, vspec],
        out_specs=[nspec, nspec, nspec],
        out_shape=[jax.ShapeDtypeStruct((N, D), jnp.float32)] * 3,
    )(h, W_lin, bias, W1a, W1b, b1)


def _sc1_body(x_hbm, a_hbm, b_hbm, row_hbm, col_hbm,
              ab_hbm, scl_hbm,
              rowv, colv,
              b00, b01, b02, b03, b10, b11, b12, b13,
              scl0, scl1, sem0, sem1, wsem0, wsem1):
    bufs = ((b00, b01, b02, b03), (b10, b11, b12, b13))
    scls = (scl0, scl1)
    sems = (sem0, sem1)
    wsems = (wsem0, wsem1)
    wid = lax.axis_index("s") * NC + lax.axis_index("c")
    base = wid * EPW
    pltpu.sync_copy(row_hbm.at[pl.ds(base, EPW)], rowv)
    pltpu.sync_copy(col_hbm.at[pl.ds(base, EPW)], colv)

    def fire(k, s, first):
        if not first:
            pltpu.make_async_copy(bufs[s][2], ab_hbm.at[pl.ds(0, CH)],
                                  wsems[s]).wait()
            pltpu.make_async_copy(scls[s], scl_hbm.at[pl.ds(0, CH)],
                                  wsems[s]).wait()
        o = k * CH
        ri = rowv.at[pl.ds(o, CH)]
        ci = colv.at[pl.ds(o, CH)]
        pltpu.async_copy(x_hbm.at[ri], bufs[s][0], sems[s])
        pltpu.async_copy(x_hbm.at[ci], bufs[s][1], sems[s])
        pltpu.async_copy(a_hbm.at[ri], bufs[s][2], sems[s])
        pltpu.async_copy(b_hbm.at[ci], bufs[s][3], sems[s])

    def drain(k, s):
        for b in range(4):
            pltpu.make_async_copy(x_hbm.at[pl.ds(0, CH)], bufs[s][b],
                                  sems[s]).wait()

        def ebody(i, _s=s):
            b0, b1, b2, b3 = bufs[_s]
            z16 = jnp.zeros((16,), jnp.float32)
            dacc, xacc, yacc = z16, z16, z16
            for j in range(8):
                sl = pl.ds(j * 16, 16)
                v1 = b0[i, sl]
                v2 = b1[i, sl]
                dacc = dacc + v1 * v2
                xacc = xacc + v1 * v1
                yacc = yacc + v2 * v2
                b2[i, sl] = b2[i, sl] + b3[i, sl]
            scls[_s][i, pl.ds(0, 16)] = dacc
            scls[_s][i, pl.ds(16, 16)] = xacc
            scls[_s][i, pl.ds(32, 16)] = yacc

        plsc.parallel_loop(0, CH, unroll=4)(ebody)

        off = base + k * CH
        pltpu.async_copy(bufs[s][2], ab_hbm.at[pl.ds(off, CH)], wsems[s])
        pltpu.async_copy(scls[s], scl_hbm.at[pl.ds(off, CH)], wsems[s])

    fire(0, 0, True)
    fire(1, 1, True)

    def body(j, carry):
        k0 = 2 * j
        drain(k0, 0)
        fire(k0 + 2, 0, False)
        drain(k0 + 1, 1)
        fire(k0 + 3, 1, False)
        return carry

    lax.fori_loop(0, (NCHUNK - 3) // 2, body, 0)
    drain(NCHUNK - 3, 0)
    fire(NCHUNK - 1, 0, False)
    drain(NCHUNK - 2, 1)
    drain(NCHUNK - 1, 0)
    for s in range(2):
        pltpu.make_async_copy(bufs[s][2], ab_hbm.at[pl.ds(0, CH)],
                              wsems[s]).wait()
        pltpu.make_async_copy(scls[s], scl_hbm.at[pl.ds(0, CH)],
                              wsems[s]).wait()


def _sc1_call(x, A, B, row, col):
    mesh = plsc.VectorSubcoreMesh(core_axis_name="c", subcore_axis_name="s")
    f = pl.kernel(
        _sc1_body,
        mesh=mesh,
        out_type=[jax.ShapeDtypeStruct((E, D), jnp.float32),
                  jax.ShapeDtypeStruct((E, 64), jnp.float32)],
        scratch_types=[pltpu.VMEM((EPW,), jnp.int32),
                       pltpu.VMEM((EPW,), jnp.int32)]
                      + [pltpu.VMEM((CH, D), jnp.float32)] * 8
                      + [pltpu.VMEM((CH, 64), jnp.float32)] * 2
                      + [pltpu.SemaphoreType.DMA] * 4,
    )
    return f(x, A, B, row, col)


def _tc2_body(ab_ref, sp_ref, d_ref, em_ref, u_ref, v_ref, w2c_ref,
              b2_ref, m_ref, g_ref, f_ref):
    s8t = lax.dot_general(m_ref[...], sp_ref[...], (((0,), (1,)), ((), ())),
                          preferred_element_type=jnp.float32)
    dot = s8t[0:1]
    x2 = s8t[1:2]
    y2 = s8t[2:3]
    alpha = 1.0 - 2.0 * dot + y2
    beta = 1.0 - x2
    num2 = alpha * alpha * x2 + beta * beta * y2 - 2.0 * alpha * beta * dot
    den = jnp.maximum(1.0 - 2.0 * dot + x2 * y2, MIN)
    subn = jnp.maximum(jnp.sqrt(jnp.maximum(num2, 0.0)) / den, MIN)
    w = _artanh(subn)
    geo = 4.0 * w * w
    scal0 = jnp.maximum(beta, MIN) * w / (den * subn) * 0.01
    ga0 = scal0 * alpha
    fb0 = scal0 * beta
    pack = jnp.concatenate([geo, ga0, fb0, geo, geo, geo, geo, geo], axis=0)
    packt = jnp.transpose(pack)
    geoc = packt[:, 0:1]
    ga0c = packt[:, 1:2]
    fb0c = packt[:, 2:3]
    z = ab_ref[...] + d_ref[...] * u_ref[...] + geoc * v_ref[...]
    mid = z * jax.nn.sigmoid(z)
    sv = jnp.dot(mid, w2c_ref[...],
                 preferred_element_type=jnp.float32) + b2_ref[0, 0]
    att = jax.nn.sigmoid(sv) * em_ref[...]
    g_ref[...] = -att * ga0c
    f_ref[...] = att * fb0c


def _tc2_call(ab, scl, distances, edge_mask, u, v, w2c, b2, M64):
    espec = pl.BlockSpec((BE, D), lambda i: (i, 0))
    pspec = pl.BlockSpec((BE, 64), lambda i: (i, 0))
    sspec = pl.BlockSpec((BE, 1), lambda i: (i, 0))
    vspec = pl.BlockSpec((1, D), lambda i: (0, 0))
    cspec = pl.BlockSpec((D, 1), lambda i: (0, 0))
    mspec = pl.BlockSpec((64, 8), lambda i: (0, 0))
    return pl.pallas_call(
        _tc2_body,
        grid=(E // BE,),
        in_specs=[espec, pspec, sspec, sspec, vspec, vspec, cspec,
                  vspec, mspec],
        out_specs=[sspec, sspec],
        out_shape=[jax.ShapeDtypeStruct((E, 1), jnp.float32)] * 2,
    )(ab, scl, distances, edge_mask, u, v, w2c, b2, M64)


def _sc2_body(x_hbm, row_hbm, col_hbm, g_hbm, f_hbm, zero_hbm, out_hbm,
              ri0, ri1, sr0, sr1, p10, p20, p11, p21, gb0, gb1, fb0, fb1, acc,
              sem0, sem1, isem0, isem1):
    srow = (sr0, sr1)
    cid = lax.axis_index("c")
    sid = lax.axis_index("s")
    wid = sid * NC + cid
    rib = (ri0, ri1)
    p1b = (p10, p11)
    p2b = (p20, p21)
    gb = (gb0, gb1)
    fb = (fb0, fb1)
    sems = (sem0, sem1)
    isems = (isem0, isem1)
    rows = 624
    pltpu.sync_copy(zero_hbm.at[pl.ds(sid * rows, rows)],
                    acc.at[pl.ds(sid * rows, rows)])

    @pl.when(sid == 0)
    def _():
        pltpu.sync_copy(zero_hbm.at[pl.ds(NS * rows, N - NS * rows)],
                        acc.at[pl.ds(NS * rows, N - NS * rows)])

    plsc.subcore_barrier()

    base = wid * EPW

    def fire_idx(k, s):
        off = base + k * CH
        pltpu.async_copy(row_hbm.at[pl.ds(off, CH)], rib[s].at[0], isems[s])
        pltpu.async_copy(col_hbm.at[pl.ds(off, CH)], rib[s].at[1], isems[s])

    def wait_idx(s):
        pltpu.make_async_copy(row_hbm.at[pl.ds(0, CH)], rib[s].at[0],
                              isems[s]).wait()
        pltpu.make_async_copy(row_hbm.at[pl.ds(0, CH)], rib[s].at[1],
                              isems[s]).wait()

    def fire(k, s):
        wait_idx(s)
        pltpu.async_copy(x_hbm.at[rib[s].at[0]], p1b[s], sems[s])
        pltpu.async_copy(x_hbm.at[rib[s].at[1]], p2b[s], sems[s])
        o = base + k * CH
        pltpu.async_copy(g_hbm.at[pl.ds(o, CH)], gb[s].at[pl.ds(0, CH)],
                         sems[s])
        pltpu.async_copy(f_hbm.at[pl.ds(o, CH)], fb[s].at[pl.ds(0, CH)],
                         sems[s])

    def drain(k, s):
        pltpu.make_async_copy(x_hbm.at[pl.ds(0, CH)], p1b[s], sems[s]).wait()
        pltpu.make_async_copy(x_hbm.at[pl.ds(0, CH)], p2b[s], sems[s]).wait()
        pltpu.make_async_copy(g_hbm.at[pl.ds(0, CH)],
                              gb[s].at[pl.ds(0, CH)], sems[s]).wait()
        pltpu.make_async_copy(f_hbm.at[pl.ds(0, CH)],
                              fb[s].at[pl.ds(0, CH)], sems[s]).wait()
        for tt in range(CH // 16):
            srow[s][pl.ds(16 * tt, 16)] = rib[s][0, pl.ds(16 * tt, 16)]
        fire_idx(jnp.minimum(k + 2, NCHUNK - 1), s)

        def mbody(i, _s=s):
            gv = gb[_s][pl.ds(i, 16)]
            fv = fb[_s][pl.ds(i, 16)]
            gs = jnp.full((16,), gv[0], jnp.float32)
            fs = jnp.full((16,), fv[0], jnp.float32)
            for j in range(8):
                sl = pl.ds(j * 16, 16)
                p1b[_s][i, sl] = gs * p1b[_s][i, sl] + fs * p2b[_s][i, sl]

        plsc.parallel_loop(0, CH, unroll=4)(mbody)
        pltpu.sync_copy(p1b[s], acc.at[srow[s]], add=True)

    fire_idx(0, 0)
    fire_idx(1, 1)
    fire(0, 0)

    def body(j, carry):
        k0 = 2 * j
        fire(k0 + 1, 1)
        drain(k0, 0)
        fire(k0 + 2, 0)
        drain(k0 + 1, 1)
        return carry

    lax.fori_loop(0, (NCHUNK - 1) // 2, body, 0)
    drain(NCHUNK - 1, 0)
    wait_idx(0)
    wait_idx(1)

    plsc.subcore_barrier()

    @pl.when(sid == 0)
    def _():
        pltpu.sync_copy(acc, out_hbm.at[cid])


def _sc2_call(x, row, col, g, f, zero):
    mesh = plsc.VectorSubcoreMesh(core_axis_name="c", subcore_axis_name="s")
    fn = pl.kernel(
        _sc2_body,
        mesh=mesh,
        out_type=jax.ShapeDtypeStruct((NC, N, D), jnp.float32),
        scratch_types=[pltpu.VMEM((2, CH), jnp.int32)] * 2
                      + [pltpu.VMEM((CH,), jnp.int32)] * 2
                      + [pltpu.VMEM((CH, D), jnp.float32)] * 4
                      + [pltpu.VMEM((CH + 16,), jnp.float32)] * 4
                      + [pltpu.VMEM_SHARED((N, D), jnp.float32)]
                      + [pltpu.SemaphoreType.DMA] * 4,
    )
    return fn(x, row, col, g, f, zero)


def _tc4_body(p_ref, x_ref, g_ref, bta_ref, o_ref):
    out = p_ref[0] + p_ref[1]
    x = x_ref[...]
    un = _rnorm(out)
    x2 = jnp.sum(x * x, axis=-1, keepdims=True)
    lam = 2.0 / jnp.maximum(1.0 - x2, MIN)
    second = jnp.tanh(0.5 * lam * un) * out / un
    x1 = _proj(_mobius_add(x, second))
    t = _logmap0(x1)
    mu = jnp.mean(t, axis=-1, keepdims=True)
    var = jnp.mean((t - mu) ** 2, axis=-1, keepdims=True)
    t = (t - mu) / jnp.sqrt(var + 1e-5) * g_ref[...] + bta_ref[...]
    x2e = _expmap0(t)
    xt = _logmap0(x2e)
    xt = xt * jax.nn.sigmoid(xt)
    o_ref[...] = _expmap0(xt)


def _tc4_call(parts, x, ln_gamma, ln_beta):
    nspec = pl.BlockSpec((BN, D), lambda i: (i, 0))
    pspec = pl.BlockSpec((NC, BN, D), lambda i: (0, i, 0))
    vspec = pl.BlockSpec((1, D), lambda i: (0, 0))
    return pl.pallas_call(
        _tc4_body,
        grid=(N // BN,),
        in_specs=[pspec, nspec, vspec, vspec],
        out_specs=nspec,
        out_shape=jax.ShapeDtypeStruct((N, D), jnp.float32),
    )(parts, x, ln_gamma, ln_beta)


def kernel(h, distances, edges, node_mask, edge_mask, W_lin, bias,
           ln_gamma, ln_beta, att_W1, att_b1, att_W2, att_b2):
    row = edges[0]
    col = edges[1]
    W1a = att_W1[:, :D]
    W1b = att_W1[:, D:2 * D]
    u = att_W1[:, 2 * D].reshape(1, D)
    v = att_W1[:, 2 * D + 1].reshape(1, D)
    w2c = att_W2.reshape(D, 1)
    b1 = att_b1.reshape(1, D)
    b2 = jnp.broadcast_to(att_b2.reshape(1, 1), (1, D))

    m_np = np.zeros((64, 8), np.float32)
    m_np[0:16, 0] = 1.0
    m_np[16:32, 1] = 1.0
    m_np[32:48, 2] = 1.0
    M64 = jnp.asarray(m_np)

    x, A, B = _tc1_call(h, W_lin, bias, W1a, W1b, b1)
    ab, scl = _sc1_call(x, A, B, row, col)
    g2, f2 = _tc2_call(ab, scl, distances, edge_mask, u, v, w2c, b2, M64)
    zero = jnp.zeros((N, D), jnp.float32)
    parts = _sc2_call(x, row, col, g2.reshape(E), f2.reshape(E), zero)
    xf = _tc4_call(parts, x, ln_gamma.reshape(1, D), ln_beta.reshape(1, D))
    return (xf, distances, edges, node_mask, edge_mask)


# speedup vs baseline: 2.4549x; 1.0229x over previous
import jax
import jax.numpy as jnp
import numpy as np
from jax import lax
from jax.experimental import pallas as pl
from jax.experimental.pallas import tpu as pltpu
from jax.experimental.pallas import tpu_sc as plsc

N = 10000
E = 320000
D = 128
MIN = 1e-15
MAXNORM = 1.0 - 4e-3

NC = 2
NS = 16
NW = NC * NS
EPW = E // NW
CH = 80
NCHUNK = EPW // CH

BN = 1000
BE = 8000
EL = E // D


def _artanh(x):
    x = jnp.clip(x, -1.0 + 1e-7, 1.0 - 1e-7)
    return 0.5 * jnp.log((1.0 + x) / (1.0 - x))


def _rnorm(x):
    return jnp.maximum(jnp.sqrt(jnp.sum(x * x, axis=-1, keepdims=True)), MIN)


def _proj(x):
    n = _rnorm(x)
    return jnp.where(n > MAXNORM, x / n * MAXNORM, x)


def _expmap0(u):
    un = _rnorm(u)
    return _proj(jnp.tanh(un) * u / un)


def _logmap0(p):
    pn = _rnorm(p)
    return p / pn * _artanh(pn)


def _mobius_add(x, y):
    x2 = jnp.sum(x * x, axis=-1, keepdims=True)
    y2 = jnp.sum(y * y, axis=-1, keepdims=True)
    xy = jnp.sum(x * y, axis=-1, keepdims=True)
    num = (1.0 + 2.0 * xy + y2) * x + (1.0 - x2) * y
    den = 1.0 + 2.0 * xy + x2 * y2
    return num / jnp.maximum(den, MIN)


def _dotT(a, b):
    return lax.dot_general(a, b, (((1,), (1,)), ((), ())),
                           preferred_element_type=jnp.float32)


def _tc1_body(h_ref, wlin_ref, bias_ref, w1a_ref, w1b_ref, b1_ref,
              x_ref, a_ref, b_ref):
    h = h_ref[...]
    x1 = _logmap0(h)
    xw = _dotT(x1, wlin_ref[...])
    xe = _expmap0(xw)
    hb = _expmap0(bias_ref[...])
    x = _mobius_add(xe, hb)
    x_ref[...] = x
    x_t = _logmap0(x)
    a_ref[...] = _dotT(x_t, w1a_ref[...]) + b1_ref[...]
    b_ref[...] = _dotT(x_t, w1b_ref[...])


def _tc1_call(h, W_lin, bias, W1a, W1b, b1):
    nspec = pl.BlockSpec((BN, D), lambda i: (i, 0))
    wspec = pl.BlockSpec((D, D), lambda i: (0, 0))
    vspec = pl.BlockSpec((1, D), lambda i: (0, 0))
    return pl.pallas_call(
        _tc1_body,
        grid=(N // BN,),
        in_specs=[nspec, wspec, vspec, wspec, wspec, vspec],
        out_specs=[nspec, nspec, nspec],
        out_shape=[jax.ShapeDtypeStruct((N, D), jnp.float32)] * 3,
    )(h, W_lin, bias, W1a, W1b, b1)


def _sc1_body(x_hbm, a_hbm, b_hbm, row_hbm, col_hbm,
              ab_hbm, scl_hbm,
              rowv, colv,
              b00, b01, b02, b03, b10, b11, b12, b13,
              scl0, scl1, sem0, sem1, wsem0, wsem1):
    bufs = ((b00, b01, b02, b03), (b10, b11, b12, b13))
    scls = (scl0, scl1)
    sems = (sem0, sem1)
    wsems = (wsem0, wsem1)
    wid = lax.axis_index("s") * NC + lax.axis_index("c")
    base = wid * EPW
    pltpu.sync_copy(row_hbm.at[pl.ds(base, EPW)], rowv)
    pltpu.sync_copy(col_hbm.at[pl.ds(base, EPW)], colv)

    def fire(k, s, first):
        if not first:
            pltpu.make_async_copy(bufs[s][2], ab_hbm.at[pl.ds(0, CH)],
                                  wsems[s]).wait()
            pltpu.make_async_copy(scls[s], scl_hbm.at[pl.ds(0, CH)],
                                  wsems[s]).wait()
        o = k * CH
        ri = rowv.at[pl.ds(o, CH)]
        ci = colv.at[pl.ds(o, CH)]
        pltpu.async_copy(x_hbm.at[ri], bufs[s][0], sems[s])
        pltpu.async_copy(x_hbm.at[ci], bufs[s][1], sems[s])
        pltpu.async_copy(a_hbm.at[ri], bufs[s][2], sems[s])
        pltpu.async_copy(b_hbm.at[ci], bufs[s][3], sems[s])

    def drain(k, s):
        for b in range(4):
            pltpu.make_async_copy(x_hbm.at[pl.ds(0, CH)], bufs[s][b],
                                  sems[s]).wait()

        def ebody(i, _s=s):
            b0, b1, b2, b3 = bufs[_s]
            z16 = jnp.zeros((16,), jnp.float32)
            dacc, xacc, yacc = z16, z16, z16
            for j in range(8):
                sl = pl.ds(j * 16, 16)
                v1 = b0[i, sl]
                v2 = b1[i, sl]
                dacc = dacc + v1 * v2
                xacc = xacc + v1 * v1
                yacc = yacc + v2 * v2
                b2[i, sl] = b2[i, sl] + b3[i, sl]
            scls[_s][i, pl.ds(0, 16)] = dacc
            scls[_s][i, pl.ds(16, 16)] = xacc
            scls[_s][i, pl.ds(32, 16)] = yacc

        plsc.parallel_loop(0, CH, unroll=4)(ebody)

        off = base + k * CH
        pltpu.async_copy(bufs[s][2], ab_hbm.at[pl.ds(off, CH)], wsems[s])
        pltpu.async_copy(scls[s], scl_hbm.at[pl.ds(off, CH)], wsems[s])

    fire(0, 0, True)
    fire(1, 1, True)

    def body(j, carry):
        k0 = 2 * j
        drain(k0, 0)
        fire(k0 + 2, 0, False)
        drain(k0 + 1, 1)
        fire(k0 + 3, 1, False)
        return carry

    lax.fori_loop(0, (NCHUNK - 3) // 2, body, 0)
    drain(NCHUNK - 3, 0)
    fire(NCHUNK - 1, 0, False)
    drain(NCHUNK - 2, 1)
    drain(NCHUNK - 1, 0)
    for s in range(2):
        pltpu.make_async_copy(bufs[s][2], ab_hbm.at[pl.ds(0, CH)],
                              wsems[s]).wait()
        pltpu.make_async_copy(scls[s], scl_hbm.at[pl.ds(0, CH)],
                              wsems[s]).wait()


def _sc1_call(x, A, B, row, col):
    mesh = plsc.VectorSubcoreMesh(core_axis_name="c", subcore_axis_name="s")
    f = pl.kernel(
        _sc1_body,
        mesh=mesh,
        out_type=[jax.ShapeDtypeStruct((E, D), jnp.float32),
                  jax.ShapeDtypeStruct((E, 64), jnp.float32)],
        scratch_types=[pltpu.VMEM((EPW,), jnp.int32),
                       pltpu.VMEM((EPW,), jnp.int32)]
                      + [pltpu.VMEM((CH, D), jnp.float32)] * 8
                      + [pltpu.VMEM((CH, 64), jnp.float32)] * 2
                      + [pltpu.SemaphoreType.DMA] * 4,
    )
    return f(x, A, B, row, col)


def _tc2_body(ab_ref, sp_ref, d_ref, em_ref, u_ref, v_ref, w2c_ref,
              b2_ref, m_ref, g_ref, f_ref):
    s8t = lax.dot_general(m_ref[...], sp_ref[...], (((0,), (1,)), ((), ())),
                          preferred_element_type=jnp.float32)
    dot = s8t[0:1]
    x2 = s8t[1:2]
    y2 = s8t[2:3]
    alpha = 1.0 - 2.0 * dot + y2
    beta = 1.0 - x2
    num2 = alpha * alpha * x2 + beta * beta * y2 - 2.0 * alpha * beta * dot
    den = jnp.maximum(1.0 - 2.0 * dot + x2 * y2, MIN)
    subn = jnp.maximum(jnp.sqrt(jnp.maximum(num2, 0.0)) / den, MIN)
    w = _artanh(subn)
    geo = 4.0 * w * w
    scal0 = jnp.maximum(beta, MIN) * w / (den * subn) * 0.01
    ga0 = scal0 * alpha
    fb0 = scal0 * beta
    pack = jnp.concatenate([geo, ga0, fb0, geo, geo, geo, geo, geo], axis=0)
    packt = jnp.transpose(pack)
    geoc = packt[:, 0:1]
    ga0c = packt[:, 1:2]
    fb0c = packt[:, 2:3]
    z = ab_ref[...] + d_ref[...] * u_ref[...] + geoc * v_ref[...]
    mid = z * jax.nn.sigmoid(z)
    sv = jnp.dot(mid, w2c_ref[...],
                 preferred_element_type=jnp.float32) + b2_ref[0, 0]
    att = jax.nn.sigmoid(sv) * em_ref[...]
    g_ref[...] = -att * ga0c
    f_ref[...] = att * fb0c


def _tc2_call(ab, scl, distances, edge_mask, u, v, w2c, b2, M64):
    espec = pl.BlockSpec((BE, D), lambda i: (i, 0))
    pspec = pl.BlockSpec((BE, 64), lambda i: (i, 0))
    sspec = pl.BlockSpec((BE, 1), lambda i: (i, 0))
    vspec = pl.BlockSpec((1, D), lambda i: (0, 0))
    cspec = pl.BlockSpec((D, 1), lambda i: (0, 0))
    mspec = pl.BlockSpec((64, 8), lambda i: (0, 0))
    return pl.pallas_call(
        _tc2_body,
        grid=(E // BE,),
        in_specs=[espec, pspec, sspec, sspec, vspec, vspec, cspec,
                  vspec, mspec],
        out_specs=[sspec, sspec],
        out_shape=[jax.ShapeDtypeStruct((E, 1), jnp.float32)] * 2,
    )(ab, scl, distances, edge_mask, u, v, w2c, b2, M64)


def _sc2_body(x_hbm, row_hbm, col_hbm, g_hbm, f_hbm, zero_hbm, out_hbm,
              ri0, ri1, sr0, sr1, p10, p20, p11, p21, gb0, gb1, fb0, fb1, acc,
              sem0, sem1, isem0, isem1):
    srow = (sr0, sr1)
    cid = lax.axis_index("c")
    sid = lax.axis_index("s")
    wid = sid * NC + cid
    rib = (ri0, ri1)
    p1b = (p10, p11)
    p2b = (p20, p21)
    gb = (gb0, gb1)
    fb = (fb0, fb1)
    sems = (sem0, sem1)
    isems = (isem0, isem1)
    rows = 624
    pltpu.sync_copy(zero_hbm.at[pl.ds(sid * rows, rows)],
                    acc.at[pl.ds(sid * rows, rows)])

    @pl.when(sid == 0)
    def _():
        pltpu.sync_copy(zero_hbm.at[pl.ds(NS * rows, N - NS * rows)],
                        acc.at[pl.ds(NS * rows, N - NS * rows)])

    plsc.subcore_barrier()

    base = wid * EPW

    def fire_idx(k, s):
        off = base + k * CH
        pltpu.async_copy(row_hbm.at[pl.ds(off, CH)], rib[s].at[0], isems[s])
        pltpu.async_copy(col_hbm.at[pl.ds(off, CH)], rib[s].at[1], isems[s])

    def wait_idx(s):
        pltpu.make_async_copy(row_hbm.at[pl.ds(0, CH)], rib[s].at[0],
                              isems[s]).wait()
        pltpu.make_async_copy(row_hbm.at[pl.ds(0, CH)], rib[s].at[1],
                              isems[s]).wait()

    def fire(k, s):
        wait_idx(s)
        pltpu.async_copy(x_hbm.at[rib[s].at[0]], p1b[s], sems[s])
        pltpu.async_copy(x_hbm.at[rib[s].at[1]], p2b[s], sems[s])
        o = base + k * CH
        pltpu.async_copy(g_hbm.at[pl.ds(o, CH)], gb[s].at[pl.ds(0, CH)],
                         sems[s])
        pltpu.async_copy(f_hbm.at[pl.ds(o, CH)], fb[s].at[pl.ds(0, CH)],
                         sems[s])

    def drain(k, s):
        pltpu.make_async_copy(x_hbm.at[pl.ds(0, CH)], p1b[s], sems[s]).wait()
        pltpu.make_async_copy(x_hbm.at[pl.ds(0, CH)], p2b[s], sems[s]).wait()
        pltpu.make_async_copy(g_hbm.at[pl.ds(0, CH)],
                              gb[s].at[pl.ds(0, CH)], sems[s]).wait()
        pltpu.make_async_copy(f_hbm.at[pl.ds(0, CH)],
                              fb[s].at[pl.ds(0, CH)], sems[s]).wait()
        for tt in range(CH // 16):
            srow[s][pl.ds(16 * tt, 16)] = rib[s][0, pl.ds(16 * tt, 16)]
        fire_idx(jnp.minimum(k + 2, NCHUNK - 1), s)

        def mbody(i, _s=s):
            gv = gb[_s][pl.ds(i, 16)]
            fv = fb[_s][pl.ds(i, 16)]
            gs = jnp.full((16,), gv[0], jnp.float32)
            fs = jnp.full((16,), fv[0], jnp.float32)
            for j in range(8):
                sl = pl.ds(j * 16, 16)
                p1b[_s][i, sl] = gs * p1b[_s][i, sl] + fs * p2b[_s][i, sl]

        plsc.parallel_loop(0, CH, unroll=4)(mbody)
        pltpu.sync_copy(p1b[s], acc.at[srow[s]], add=True)

    fire_idx(0, 0)
    fire_idx(1, 1)
    fire(0, 0)

    def body(j, carry):
        k0 = 2 * j
        fire(k0 + 1, 1)
        drain(k0, 0)
        fire(k0 + 2, 0)
        drain(k0 + 1, 1)
        return carry

    lax.fori_loop(0, (NCHUNK - 1) // 2, body, 0)
    drain(NCHUNK - 1, 0)
    wait_idx(0)
    wait_idx(1)

    plsc.subcore_barrier()

    @pl.when(sid == 0)
    def _():
        pltpu.sync_copy(acc, out_hbm.at[cid])


def _sc2_call(x, row, col, g, f, zero):
    mesh = plsc.VectorSubcoreMesh(core_axis_name="c", subcore_axis_name="s")
    fn = pl.kernel(
        _sc2_body,
        mesh=mesh,
        out_type=jax.ShapeDtypeStruct((NC, N, D), jnp.float32),
        scratch_types=[pltpu.VMEM((2, CH), jnp.int32)] * 2
                      + [pltpu.VMEM((CH,), jnp.int32)] * 2
                      + [pltpu.VMEM((CH, D), jnp.float32)] * 4
                      + [pltpu.VMEM((CH + 16,), jnp.float32)] * 4
                      + [pltpu.VMEM_SHARED((N, D), jnp.float32)]
                      + [pltpu.SemaphoreType.DMA] * 4,
    )
    return fn(x, row, col, g, f, zero)


def _tc4_body(p_ref, x_ref, g_ref, bta_ref, o_ref):
    out = p_ref[0] + p_ref[1]
    x = x_ref[...]
    un = _rnorm(out)
    x2 = jnp.sum(x * x, axis=-1, keepdims=True)
    lam = 2.0 / jnp.maximum(1.0 - x2, MIN)
    second = jnp.tanh(0.5 * lam * un) * out / un
    x1 = _proj(_mobius_add(x, second))
    t = _logmap0(x1)
    mu = jnp.mean(t, axis=-1, keepdims=True)
    var = jnp.mean((t - mu) ** 2, axis=-1, keepdims=True)
    t = (t - mu) / jnp.sqrt(var + 1e-5) * g_ref[...] + bta_ref[...]
    x2e = _expmap0(t)
    xt = _logmap0(x2e)
    xt = xt * jax.nn.sigmoid(xt)
    o_ref[...] = _expmap0(xt)


def _tc4_call(parts, x, ln_gamma, ln_beta):
    nspec = pl.BlockSpec((BN, D), lambda i: (i, 0))
    pspec = pl.BlockSpec((NC, BN, D), lambda i: (0, i, 0))
    vspec = pl.BlockSpec((1, D), lambda i: (0, 0))
    return pl.pallas_call(
        _tc4_body,
        grid=(N // BN,),
        in_specs=[pspec, nspec, vspec, vspec],
        out_specs=nspec,
        out_shape=jax.ShapeDtypeStruct((N, D), jnp.float32),
    )(parts, x, ln_gamma, ln_beta)


def kernel(h, distances, edges, node_mask, edge_mask, W_lin, bias,
           ln_gamma, ln_beta, att_W1, att_b1, att_W2, att_b2):
    row = edges[0]
    col = edges[1]
    W1a = att_W1[:, :D]
    W1b = att_W1[:, D:2 * D]
    u = att_W1[:, 2 * D].reshape(1, D)
    v = att_W1[:, 2 * D + 1].reshape(1, D)
    w2c = att_W2.reshape(D, 1)
    b1 = att_b1.reshape(1, D)
    b2 = jnp.broadcast_to(att_b2.reshape(1, 1), (1, D))

    m_np = np.zeros((64, 8), np.float32)
    m_np[0:16, 0] = 1.0
    m_np[16:32, 1] = 1.0
    m_np[32:48, 2] = 1.0
    M64 = jnp.asarray(m_np)

    x, A, B = _tc1_call(h, W_lin, bias, W1a, W1b, b1)
    ab, scl = _sc1_call(x, A, B, row, col)
    g2, f2 = _tc2_call(ab, scl, distances, edge_mask, u, v, w2c, b2, M64)
    zero = jnp.zeros((N, D), jnp.float32)
    parts = _sc2_call(x, row, col, g2.reshape(E), f2.reshape(E), zero)
    xf = _tc4_call(parts, x, ln_gamma.reshape(1, D), ln_beta.reshape(1, D))
    return (xf, distances, edges, node_mask, edge_mask)
